# baseline (speedup 1.0000x reference)
import functools

import jax
import jax.numpy as jnp
from jax import lax
from jax.experimental import pallas as pl
from jax.experimental.pallas import tpu as pltpu
from jax.experimental.pallas import tpu_sc as plsc

F32 = jnp.float32
I32 = jnp.int32

N = 10000
E = 160000
D = 128
EO = 512
NG = 16
EPS = 1e-5

NC = 2
NS = 16
NW = NC * NS
EB = 128
E_PAD = 163840
NBLK = E_PAD // EB
BPW = E_PAD // (NW * EB)
N_PAD = 10112
ZR = N_PAD // NS

DP = 64

TE = 640
GE = E_PAD // TE
TN = 1000
GN = N // TN


@functools.cache
def _sc_build():
    mesh = plsc.VectorSubcoreMesh(core_axis_name="c", subcore_axis_name="s")
    GD = 3
    NGRP = BPW // GD
    GDG = 5
    NGRPG = BPW // GDG

    @functools.partial(
        pl.kernel,
        out_type=jax.ShapeDtypeStruct((NBLK, 2 * EB, DP), F32),
        mesh=mesh,
        compiler_params=pltpu.CompilerParams(use_tc_tiling_on_sc=False),
        scratch_types=(
            [pltpu.VMEM((2, EB), I32) for _ in range(GDG)]
            + [pltpu.VMEM((2 * EB, DP), F32) for _ in range(GDG)]
            + [pltpu.SemaphoreType.DMA for _ in range(2 * GDG)]
        ),
    )
    def gather2(x_hbm, idx2_hbm, out_hbm, *scr):
        ib = scr[0:GDG]
        bd = scr[GDG:2 * GDG]
        sems = scr[2 * GDG:4 * GDG]
        wid = lax.axis_index("c") * NS + lax.axis_index("s")
        base = wid * BPW

        def do_group(first_blk, nk):
            for k in range(nk):
                g = base + first_blk + k
                pltpu.sync_copy(idx2_hbm.at[g], ib[k])
                pltpu.async_copy(x_hbm.at[ib[k].at[0]], bd[k].at[pl.ds(0, EB)],
                                 sems[2 * k])
                pltpu.async_copy(x_hbm.at[ib[k].at[1]], bd[k].at[pl.ds(EB, EB)],
                                 sems[2 * k + 1])
            for k in range(nk):
                g = base + first_blk + k
                pltpu.make_async_copy(x_hbm.at[ib[k].at[0]],
                                      bd[k].at[pl.ds(0, EB)], sems[2 * k]).wait()
                pltpu.make_async_copy(x_hbm.at[ib[k].at[1]],
                                      bd[k].at[pl.ds(EB, EB)], sems[2 * k + 1]).wait()
                pltpu.sync_copy(bd[k], out_hbm.at[g])

        def body(i, carry):
            do_group(i * GDG, GDG)
            return carry

        lax.fori_loop(0, NGRPG, body, 0)
        if BPW % GDG:
            do_group(NGRPG * GDG, BPW % GDG)

    @functools.partial(
        pl.kernel,
        out_type=jax.ShapeDtypeStruct((NC, N_PAD, D), F32),
        mesh=mesh,
        scratch_types=(
            [pltpu.VMEM((EB,), I32) for _ in range(GD)]
            + [pltpu.VMEM((EB, D), F32) for _ in range(GD)]
            + [pltpu.VMEM_SHARED((N_PAD, D), F32)]
            + [pltpu.SemaphoreType.DMA for _ in range(2 * GD)]
        ),
    )
    def scatter_add(m_hbm, col_hbm, zeros_hbm, out_hbm, *scr):
        idx = scr[0:GD]
        mb = scr[GD:2 * GD]
        acc_sh = scr[2 * GD]
        lsem = scr[2 * GD + 1:2 * GD + 1 + GD]
        ssem = scr[2 * GD + 1 + GD:2 * GD + 1 + 2 * GD]
        c = lax.axis_index("c")
        s = lax.axis_index("s")
        base = (c * NS + s) * BPW

        pltpu.sync_copy(zeros_hbm, mb[0])
        zoff = 0
        while zoff < ZR:
            step = min(EB, ZR - zoff)
            pltpu.sync_copy(mb[0].at[pl.ds(0, step)],
                            acc_sh.at[pl.ds(s * ZR + zoff, step)])
            zoff += step
        plsc.subcore_barrier()

        def fire_loads(k, blk):
            off = pl.multiple_of((base + blk) * EB, EB)
            pltpu.async_copy(col_hbm.at[pl.ds(off, EB)], idx[k], lsem[k])
            pltpu.async_copy(m_hbm.at[pl.ds(off, EB)], mb[k], lsem[k])

        def wait_loads(k, blk):
            off = pl.multiple_of((base + blk) * EB, EB)
            pltpu.make_async_copy(col_hbm.at[pl.ds(off, EB)], idx[k], lsem[k]).wait()
            pltpu.make_async_copy(m_hbm.at[pl.ds(off, EB)], mb[k], lsem[k]).wait()

        for k in range(GD):
            fire_loads(k, k)

        def body(i, carry):
            for k in range(GD):
                wait_loads(k, i * GD + k)
                pltpu.async_copy(mb[k], acc_sh.at[idx[k]], ssem[k], add=True)
            for k in range(GD):
                pltpu.make_async_copy(mb[k], acc_sh.at[idx[k]], ssem[k]).wait()

                @pl.when(i < NGRP - 1)
                def _():
                    fire_loads(k, (i + 1) * GD + k)
            return carry

        lax.fori_loop(0, NGRP, body, 0)
        off = pl.multiple_of((base + NGRP * GD) * EB, EB)
        pltpu.sync_copy(col_hbm.at[pl.ds(off, EB)], idx[0])
        pltpu.sync_copy(m_hbm.at[pl.ds(off, EB)], mb[0])
        pltpu.sync_copy(mb[0], acc_sh.at[idx[0]], add=True)

        plsc.subcore_barrier()
        zoff = 0
        while zoff < ZR:
            step = min(EB, ZR - zoff)
            pltpu.sync_copy(acc_sh.at[pl.ds(s * ZR + zoff, step)],
                            mb[0].at[pl.ds(0, step)])
            pltpu.sync_copy(mb[0].at[pl.ds(0, step)],
                            out_hbm.at[c, pl.ds(s * ZR + zoff, step)])
            zoff += step

    @functools.partial(
        pl.kernel,
        out_type=jax.ShapeDtypeStruct((NC, N_PAD, D), F32),
        mesh=mesh,
        scratch_types=(
            [pltpu.VMEM((EB,), I32) for _ in range(GD)]
            + [pltpu.VMEM((EB, D), F32)]
            + [pltpu.VMEM_SHARED((N_PAD, D), F32)]
            + [pltpu.SemaphoreType.DMA for _ in range(2 * GD)]
        ),
    )
    def count(col_hbm, ones_hbm, zeros_hbm, out_hbm, *scr):
        idx = scr[0:GD]
        ones_v = scr[GD]
        acc_sh = scr[GD + 1]
        lsem = scr[GD + 2:GD + 2 + GD]
        ssem = scr[GD + 2 + GD:GD + 2 + 2 * GD]
        c = lax.axis_index("c")
        s = lax.axis_index("s")
        base = (c * NS + s) * BPW

        pltpu.sync_copy(zeros_hbm, ones_v)
        zoff = 0
        while zoff < ZR:
            step = min(EB, ZR - zoff)
            pltpu.sync_copy(ones_v.at[pl.ds(0, step)],
                            acc_sh.at[pl.ds(s * ZR + zoff, step)])
            zoff += step
        pltpu.sync_copy(ones_hbm, ones_v)
        plsc.subcore_barrier()

        def fire_load(k, blk):
            off = pl.multiple_of((base + blk) * EB, EB)
            pltpu.async_copy(col_hbm.at[pl.ds(off, EB)], idx[k], lsem[k])

        def wait_load(k, blk):
            off = pl.multiple_of((base + blk) * EB, EB)
            pltpu.make_async_copy(col_hbm.at[pl.ds(off, EB)], idx[k], lsem[k]).wait()

        for k in range(GD):
            fire_load(k, k)

        def body(i, carry):
            for k in range(GD):
                wait_load(k, i * GD + k)
                pltpu.async_copy(ones_v, acc_sh.at[idx[k]], ssem[k], add=True)
            for k in range(GD):
                pltpu.make_async_copy(ones_v, acc_sh.at[idx[k]], ssem[k]).wait()

                @pl.when(i < NGRP - 1)
                def _():
                    fire_load(k, (i + 1) * GD + k)
            return carry

        lax.fori_loop(0, NGRP, body, 0)
        off = pl.multiple_of((base + NGRP * GD) * EB, EB)
        pltpu.sync_copy(col_hbm.at[pl.ds(off, EB)], idx[0])
        pltpu.sync_copy(ones_v, acc_sh.at[idx[0]], add=True)

        plsc.subcore_barrier()
        zoff = 0
        while zoff < ZR:
            step = min(EB, ZR - zoff)
            pltpu.sync_copy(acc_sh.at[pl.ds(s * ZR + zoff, step)],
                            ones_v.at[pl.ds(0, step)])
            pltpu.sync_copy(ones_v.at[pl.ds(0, step)],
                            out_hbm.at[c, pl.ds(s * ZR + zoff, step)])
            zoff += step

    return gather2, scatter_add, count


def _sc_gather2(x, idx2):
    return _sc_build()[0](x, idx2)


def _sc_scatter_add(m, col_s, zeros):
    return _sc_build()[1](m, col_s, zeros)


def _sc_count(col_s, ones, zeros):
    return _sc_build()[2](col_s, ones, zeros)


BF16 = jnp.bfloat16
U32 = jnp.uint32
U16 = jnp.uint16


def _unpack2(p):
    u = lax.bitcast_convert_type(p, U32)
    lo = lax.bitcast_convert_type((u & 0xFFFF).astype(U16), BF16)
    hi = lax.bitcast_convert_type((u >> 16).astype(U16), BF16)
    return jnp.concatenate([lo, hi], axis=1)


def _pack2(x16):
    lo = lax.bitcast_convert_type(x16[:, :DP], U16).astype(U32)
    hi = lax.bitcast_convert_type(x16[:, DP:], U16).astype(U32)
    return lax.bitcast_convert_type(lo | (hi << 16), F32)


TEB = TE // EB


def _edge_body(xr, xd, ea, wes, wed, wee, be, wn1x, wn1e, bn1, e_ref, m_ref):
    xr16 = _unpack2(jnp.reshape(xr[...], (TE, DP)))
    xd16 = _unpack2(jnp.reshape(xd[...], (TE, DP)))
    acc = jnp.dot(xr16, wes[...], preferred_element_type=F32)
    acc += jnp.dot(xd16, wed[...], preferred_element_type=F32)
    acc += jnp.dot(ea[...], wee[...], preferred_element_type=F32)
    e16 = jnp.maximum(acc + be[...], 0.0).astype(BF16)
    e_ref[...] = e16
    m = jnp.dot(xr16, wn1x[...], preferred_element_type=F32)
    m += jnp.dot(e16, wn1e[...], preferred_element_type=F32)
    m_ref[...] = jnp.maximum(m + bn1[...], 0.0)


def _tc_edge(g2, ea, wes, wed, wee, be, wn1x, wn1e, bn1):
    ei = ea.shape[1]
    cst = lambda i: (0, 0)
    row = lambda i: (i, 0)
    return pl.pallas_call(
        _edge_body,
        grid=(GE,),
        in_specs=[
            pl.BlockSpec((TEB, EB, DP), lambda i: (i, 0, 0)),
            pl.BlockSpec((TEB, EB, DP), lambda i: (i, 1, 0)),
            pl.BlockSpec((TE, ei), row),
            pl.BlockSpec((D, EO), cst),
            pl.BlockSpec((D, EO), cst),
            pl.BlockSpec((ei, EO), cst),
            pl.BlockSpec((1, EO), cst),
            pl.BlockSpec((D, D), cst),
            pl.BlockSpec((EO, D), cst),
            pl.BlockSpec((1, D), cst),
        ],
        out_specs=[pl.BlockSpec((TE, EO), row), pl.BlockSpec((TE, D), row)],
        out_shape=[jax.ShapeDtypeStruct((E_PAD, EO), BF16),
                   jax.ShapeDtypeStruct((E_PAD, D), F32)],
    )(g2, g2, ea, wes, wed, wee, be, wn1x, wn1e, bn1)


def _node_body(x, p0, p1, invc, b3, wn2x, wn2a, bn2, xn_ref, gs_ref, gc_ref):
    i = pl.program_id(0)
    agg = (p0[0] + p1[0]) * invc[...]
    xn = jnp.dot(_unpack2(x[...]), wn2x[...], preferred_element_type=F32)
    xn += jnp.dot(agg, wn2a[...], preferred_element_type=F32)
    xn = jnp.maximum(xn + bn2[...], 0.0)
    xn_ref[...] = _pack2(xn.astype(BF16))
    seg = b3[0]
    ids = lax.broadcasted_iota(I32, (NG, TN), 0)
    oh = (ids == seg).astype(F32)

    @pl.when(i == 0)
    def _init():
        gs_ref[...] = jnp.zeros_like(gs_ref)
        gc_ref[...] = jnp.zeros_like(gc_ref)

    gs_ref[...] += jnp.dot(oh, xn, preferred_element_type=F32)
    gc_ref[...] += jnp.broadcast_to(jnp.sum(oh, axis=1, keepdims=True), (NG, D))


def _tc_node(x, parts, invc, b3, wn2x, wn2a, bn2):
    cst = lambda i: (0, 0)
    return pl.pallas_call(
        _node_body,
        grid=(GN,),
        in_specs=[
            pl.BlockSpec((TN, DP), lambda i: (i, 0)),
            pl.BlockSpec((1, TN, D), lambda i: (0, i, 0)),
            pl.BlockSpec((1, TN, D), lambda i: (1, i, 0)),
            pl.BlockSpec((TN, D), lambda i: (i, 0)),
            pl.BlockSpec((1, 1, TN), lambda i: (i, 0, 0)),
            pl.BlockSpec((D, D), cst),
            pl.BlockSpec((D, D), cst),
            pl.BlockSpec((1, D), cst),
        ],
        out_specs=[pl.BlockSpec((TN, DP), lambda i: (i, 0)),
                   pl.BlockSpec((NG, D), cst),
                   pl.BlockSpec((NG, D), cst)],
        out_shape=[jax.ShapeDtypeStruct((N, DP), F32),
                   jax.ShapeDtypeStruct((NG, D), F32),
                   jax.ShapeDtypeStruct((NG, D), F32)],
    )(x, parts, parts, invc, b3, wn2x, wn2a, bn2)


def _inv_body(c0, c1, out):
    out[...] = 1.0 / jnp.maximum(c0[0] + c1[0], 1.0)


def _tc_invcnt(cnt_parts):
    return pl.pallas_call(
        _inv_body,
        grid=(GN,),
        in_specs=[pl.BlockSpec((1, TN, D), lambda i: (0, i, 0)),
                  pl.BlockSpec((1, TN, D), lambda i: (1, i, 0))],
        out_specs=pl.BlockSpec((TN, D), lambda i: (i, 0)),
        out_shape=jax.ShapeDtypeStruct((N, D), F32),
    )(cnt_parts, cnt_parts)


def _tc_global(gs, gc, u, wgu, wgm, bg):
    go = wgm.shape[1]

    if u is None:
        def body(gs_r, gc_r, wgm_r, bg_r, out):
            gm = gs_r[...] / jnp.maximum(gc_r[...], 1.0)
            out[...] = jnp.dot(gm, wgm_r[...], preferred_element_type=F32) + bg_r[...]
        args = (gs, gc, wgm, bg)
    else:
        def body(gs_r, gc_r, u_r, wgu_r, wgm_r, bg_r, out):
            gm = gs_r[...] / jnp.maximum(gc_r[...], 1.0)
            acc = jnp.dot(gm, wgm_r[...], preferred_element_type=F32)
            acc += jnp.dot(u_r[...], wgu_r[...], preferred_element_type=F32)
            out[...] = acc + bg_r[...]
        args = (gs, gc, u, wgu, wgm, bg)

    return pl.pallas_call(
        body,
        out_shape=jax.ShapeDtypeStruct((NG, go), F32),
    )(*args)


def _head_body(rg, pg, wr, br, wp, bp, wyr, wyp, by, out):
    a = jnp.dot(rg[...], wr[...], preferred_element_type=F32) + br[...]
    a = jnp.where(a > 0, a, jnp.exp(jnp.minimum(a, 0.0)) - 1.0)
    b = jnp.dot(pg[...], wp[...], preferred_element_type=F32) + bp[...]
    b = jnp.where(b > 0, b, jnp.exp(jnp.minimum(b, 0.0)) - 1.0)
    y = jnp.dot(a, wyr[...], preferred_element_type=F32)
    y += jnp.dot(b, wyp[...], preferred_element_type=F32)
    y = jax.nn.sigmoid(y + by[...]) * 100.0
    out[...] = jnp.broadcast_to(y[:, :1], (NG, D))


def _tc_head(rg, pg, wr, br, wp, bp, wyr, wyp, by):
    return pl.pallas_call(
        _head_body,
        out_shape=jax.ShapeDtypeStruct((NG, D), F32),
    )(rg, pg, wr, br, wp, bp, wyr, wyp, by)


def _meta_layer(x, ea, u, idx2, col_s, invc, b3, zeros, prm):
    (wes, wed, wee, be, wn1x, wn1e, bn1, wn2x, wn2a, bn2, wgu, wgm, bg) = prm
    g2 = _sc_gather2(x, idx2)
    e, m = _tc_edge(g2, ea, wes, wed, wee, be, wn1x, wn1e, bn1)
    parts = _sc_scatter_add(m, col_s, zeros)
    xn, gs, gc = _tc_node(x, parts, invc, b3, wn2x, wn2a, bn2)
    un = _tc_global(gs, gc, u, wgu, wgm, bg)
    return xn, e, un


def _split_layer(We, be, Wn1, bn1, Wn2, bn2, Wg, bg, gi, fold_bn=None):
    wes, wed, wee = We[:D], We[D:2 * D], We[2 * D:]
    wn1x, wn1e = Wn1[:D], Wn1[D:]
    wn2x, wn2a = Wn2[:D], Wn2[D:]
    if fold_bn is not None:
        nsc, nsh, esc, esh = fold_bn
        be = be + nsh @ wes + nsh @ wed + esh @ wee
        wes = nsc[:, None] * wes
        wed = nsc[:, None] * wed
        wee = esc[:, None] * wee
        bn1 = bn1 + nsh @ wn1x
        wn1x = nsc[:, None] * wn1x
        bn2 = bn2 + nsh @ wn2x
        wn2x = nsc[:, None] * wn2x
    wgu, wgm = (Wg[:gi], Wg[gi:]) if gi > 0 else (None, Wg)
    b16 = jnp.bfloat16
    return (wes.astype(b16), wed.astype(b16), wee.astype(b16), be[None, :],
            wn1x.astype(b16), wn1e.astype(b16), bn1[None, :],
            wn2x.astype(b16), wn2a, bn2[None, :], wgu, wgm, bg[None, :])


def kernel(rx, re, rg, px, pe, rc, pc, rb, pb, bnn_g, bnn_b, bnn_m, bnn_v, bne_g, bne_b, bne_m, bne_v, We_r1, be_r1, Wn1_r1, bn1_r1, Wn2_r1, bn2_r1, Wg_r1, bg_r1, We_r2, be_r2, Wn1_r2, bn1_r2, Wn2_r2, bn2_r2, Wg_r2, bg_r2, We_r3, be_r3, Wn1_r3, bn1_r3, Wn2_r3, bn2_r3, Wg_r3, bg_r3, We_r4, be_r4, Wn1_r4, bn1_r4, Wn2_r4, bn2_r4, Wg_r4, bg_r4, We_r5, be_r5, Wn1_r5, bn1_r5, Wn2_r5, bn2_r5, Wg_r5, bg_r5, We_p1, be_p1, Wn1_p1, bn1_p1, Wn2_p1, bn2_p1, Wg_p1, bg_p1, We_p2, be_p2, Wn1_p2, bn1_p2, Wn2_p2, bn2_p2, Wg_p2, bg_p2, We_p3, be_p3, Wn1_p3, bn1_p3, Wn2_p3, bn2_p3, Wg_p3, bg_p3, W_rlin, b_rlin, W_plin, b_plin, W_y, b_y):
    pad = E_PAD - E
    zpad = jnp.zeros((pad,), I32)
    npad = jnp.full((pad,), N, I32)
    rrow = jnp.concatenate([rc[0].astype(I32), zpad])
    rcol_g = jnp.concatenate([rc[1].astype(I32), zpad])
    rcol_s = jnp.concatenate([rc[1].astype(I32), npad])
    prow = jnp.concatenate([pc[0].astype(I32), zpad])
    pcol_g = jnp.concatenate([pc[1].astype(I32), zpad])
    pcol_s = jnp.concatenate([pc[1].astype(I32), npad])
    ridx2 = jnp.stack([rrow.reshape(NBLK, EB), rcol_g.reshape(NBLK, EB)], axis=1)
    pidx2 = jnp.stack([prow.reshape(NBLK, EB), pcol_g.reshape(NBLK, EB)], axis=1)
    rb3 = rb.astype(I32).reshape(GN, 1, TN)
    pb3 = pb.astype(I32).reshape(GN, 1, TN)

    nsc = bnn_g / jnp.sqrt(bnn_v + EPS)
    nsh = bnn_b - bnn_m * nsc
    esc = bne_g / jnp.sqrt(bne_v + EPS)
    esh = bne_b - bne_m * esc
    fold = (nsc, nsh, esc, esh)

    re_p = jnp.concatenate([re, jnp.zeros((pad, re.shape[1]), F32)]).astype(jnp.bfloat16)
    pe_p = jnp.concatenate([pe, jnp.zeros((pad, pe.shape[1]), F32)]).astype(jnp.bfloat16)

    zeros = jnp.zeros((EB, D), F32)
    ones = jnp.ones((EB, D), F32)

    def pack_host(x):
        x16 = x.astype(jnp.bfloat16)
        lo = lax.bitcast_convert_type(x16[:, :DP], jnp.uint16).astype(jnp.uint32)
        hi = lax.bitcast_convert_type(x16[:, DP:], jnp.uint16).astype(jnp.uint32)
        return lax.bitcast_convert_type(lo | (hi << 16), F32)

    rx_p = pack_host(rx)
    px_p = pack_host(px)

    r_layers = [
        _split_layer(We_r1, be_r1, Wn1_r1, bn1_r1, Wn2_r1, bn2_r1, Wg_r1, bg_r1, 32, fold),
        _split_layer(We_r2, be_r2, Wn1_r2, bn1_r2, Wn2_r2, bn2_r2, Wg_r2, bg_r2, 128),
        _split_layer(We_r3, be_r3, Wn1_r3, bn1_r3, Wn2_r3, bn2_r3, Wg_r3, bg_r3, 128),
        _split_layer(We_r4, be_r4, Wn1_r4, bn1_r4, Wn2_r4, bn2_r4, Wg_r4, bg_r4, 128),
        _split_layer(We_r5, be_r5, Wn1_r5, bn1_r5, Wn2_r5, bn2_r5, Wg_r5, bg_r5, 128),
    ]
    p_layers = [
        _split_layer(We_p1, be_p1, Wn1_p1, bn1_p1, Wn2_p1, bn2_p1, Wg_p1, bg_p1, 0, fold),
        _split_layer(We_p2, be_p2, Wn1_p2, bn1_p2, Wn2_p2, bn2_p2, Wg_p2, bg_p2, 128),
        _split_layer(We_p3, be_p3, Wn1_p3, bn1_p3, Wn2_p3, bn2_p3, Wg_p3, bg_p3, 128),
    ]

    r_invc = _tc_invcnt(_sc_count(rcol_s, ones, zeros))
    p_invc = _tc_invcnt(_sc_count(pcol_s, ones, zeros))

    rst = (rx_p, re_p, rg)
    pst = (px_p, pe_p, None)
    for i in range(5):
        rst = _meta_layer(*rst, ridx2, rcol_s, r_invc, rb3, zeros, r_layers[i])
        if i < 3:
            pst = _meta_layer(*pst, pidx2, pcol_s, p_invc, pb3, zeros, p_layers[i])
    rg_out = rst[2]
    pg_out = pst[2]

    y = _tc_head(rg_out, pg_out,
                 W_rlin, b_rlin[None, :], W_plin, b_plin[None, :],
                 W_y[:256], W_y[256:], b_y[None, :])
    return y[:, :1]


# speedup vs baseline: 1.0303x; 1.0303x over previous
import functools

import jax
import jax.numpy as jnp
from jax import lax
from jax.experimental import pallas as pl
from jax.experimental.pallas import tpu as pltpu
from jax.experimental.pallas import tpu_sc as plsc

F32 = jnp.float32
I32 = jnp.int32

N = 10000
E = 160000
D = 128
EO = 512
NG = 16
EPS = 1e-5

NC = 2
NS = 16
NW = NC * NS
EB = 128
E_PAD = 163840
NBLK = E_PAD // EB
BPW = E_PAD // (NW * EB)
N_PAD = 10112
ZR = N_PAD // NS

DP = 64

TE = 640
GE = E_PAD // TE
TN = 1000
GN = N // TN


@functools.cache
def _sc_build():
    mesh = plsc.VectorSubcoreMesh(core_axis_name="c", subcore_axis_name="s")
    GD = 3
    NGRP = BPW // GD
    GDG = 5
    NGRPG = BPW // GDG

    @functools.partial(
        pl.kernel,
        out_type=(jax.ShapeDtypeStruct((E_PAD, DP), F32),
                  jax.ShapeDtypeStruct((E_PAD, DP), F32)),
        mesh=mesh,
        compiler_params=pltpu.CompilerParams(use_tc_tiling_on_sc=False),
        scratch_types=(
            [pltpu.VMEM((2, EB), I32) for _ in range(GDG)]
            + [pltpu.VMEM((2 * EB, DP), F32) for _ in range(GDG)]
            + [pltpu.SemaphoreType.DMA for _ in range(2 * GDG)]
        ),
    )
    def gather2(x_hbm, idx2_hbm, outr_hbm, outc_hbm, *scr):
        ib = scr[0:GDG]
        bd = scr[GDG:2 * GDG]
        sems = scr[2 * GDG:4 * GDG]
        wid = lax.axis_index("c") * NS + lax.axis_index("s")
        base = wid * BPW

        def do_group(first_blk, nk):
            for k in range(nk):
                g = base + first_blk + k
                pltpu.sync_copy(idx2_hbm.at[g], ib[k])
                pltpu.async_copy(x_hbm.at[ib[k].at[0]], bd[k].at[pl.ds(0, EB)],
                                 sems[2 * k])
                pltpu.async_copy(x_hbm.at[ib[k].at[1]], bd[k].at[pl.ds(EB, EB)],
                                 sems[2 * k + 1])
            for k in range(nk):
                g = base + first_blk + k
                off = pl.multiple_of(g * EB, EB)
                pltpu.make_async_copy(x_hbm.at[ib[k].at[0]],
                                      bd[k].at[pl.ds(0, EB)], sems[2 * k]).wait()
                pltpu.make_async_copy(x_hbm.at[ib[k].at[1]],
                                      bd[k].at[pl.ds(EB, EB)], sems[2 * k + 1]).wait()
                pltpu.sync_copy(bd[k].at[pl.ds(0, EB)], outr_hbm.at[pl.ds(off, EB)])
                pltpu.sync_copy(bd[k].at[pl.ds(EB, EB)], outc_hbm.at[pl.ds(off, EB)])

        def body(i, carry):
            do_group(i * GDG, GDG)
            return carry

        lax.fori_loop(0, NGRPG, body, 0)
        if BPW % GDG:
            do_group(NGRPG * GDG, BPW % GDG)

    @functools.partial(
        pl.kernel,
        out_type=jax.ShapeDtypeStruct((NC, N_PAD, D), F32),
        mesh=mesh,
        scratch_types=(
            [pltpu.VMEM((EB,), I32) for _ in range(GD)]
            + [pltpu.VMEM((EB, D), F32) for _ in range(GD)]
            + [pltpu.VMEM_SHARED((N_PAD, D), F32)]
            + [pltpu.SemaphoreType.DMA for _ in range(2 * GD)]
        ),
    )
    def scatter_add(m_hbm, col_hbm, zeros_hbm, out_hbm, *scr):
        idx = scr[0:GD]
        mb = scr[GD:2 * GD]
        acc_sh = scr[2 * GD]
        lsem = scr[2 * GD + 1:2 * GD + 1 + GD]
        ssem = scr[2 * GD + 1 + GD:2 * GD + 1 + 2 * GD]
        c = lax.axis_index("c")
        s = lax.axis_index("s")
        base = (c * NS + s) * BPW

        pltpu.sync_copy(zeros_hbm, mb[0])
        zoff = 0
        while zoff < ZR:
            step = min(EB, ZR - zoff)
            pltpu.sync_copy(mb[0].at[pl.ds(0, step)],
                            acc_sh.at[pl.ds(s * ZR + zoff, step)])
            zoff += step
        plsc.subcore_barrier()

        def fire_loads(k, blk):
            off = pl.multiple_of((base + blk) * EB, EB)
            pltpu.async_copy(col_hbm.at[pl.ds(off, EB)], idx[k], lsem[k])
            pltpu.async_copy(m_hbm.at[pl.ds(off, EB)], mb[k], lsem[k])

        def wait_loads(k, blk):
            off = pl.multiple_of((base + blk) * EB, EB)
            pltpu.make_async_copy(col_hbm.at[pl.ds(off, EB)], idx[k], lsem[k]).wait()
            pltpu.make_async_copy(m_hbm.at[pl.ds(off, EB)], mb[k], lsem[k]).wait()

        for k in range(GD):
            fire_loads(k, k)

        def body(i, carry):
            for k in range(GD):
                wait_loads(k, i * GD + k)
                pltpu.async_copy(mb[k], acc_sh.at[idx[k]], ssem[k], add=True)
            for k in range(GD):
                pltpu.make_async_copy(mb[k], acc_sh.at[idx[k]], ssem[k]).wait()

                @pl.when(i < NGRP - 1)
                def _():
                    fire_loads(k, (i + 1) * GD + k)
            return carry

        lax.fori_loop(0, NGRP, body, 0)
        off = pl.multiple_of((base + NGRP * GD) * EB, EB)
        pltpu.sync_copy(col_hbm.at[pl.ds(off, EB)], idx[0])
        pltpu.sync_copy(m_hbm.at[pl.ds(off, EB)], mb[0])
        pltpu.sync_copy(mb[0], acc_sh.at[idx[0]], add=True)

        plsc.subcore_barrier()
        zoff = 0
        while zoff < ZR:
            step = min(EB, ZR - zoff)
            pltpu.sync_copy(acc_sh.at[pl.ds(s * ZR + zoff, step)],
                            mb[0].at[pl.ds(0, step)])
            pltpu.sync_copy(mb[0].at[pl.ds(0, step)],
                            out_hbm.at[c, pl.ds(s * ZR + zoff, step)])
            zoff += step

    @functools.partial(
        pl.kernel,
        out_type=jax.ShapeDtypeStruct((NC, N_PAD, D), F32),
        mesh=mesh,
        scratch_types=(
            [pltpu.VMEM((EB,), I32) for _ in range(GD)]
            + [pltpu.VMEM((EB, D), F32)]
            + [pltpu.VMEM_SHARED((N_PAD, D), F32)]
            + [pltpu.SemaphoreType.DMA for _ in range(2 * GD)]
        ),
    )
    def count(col_hbm, ones_hbm, zeros_hbm, out_hbm, *scr):
        idx = scr[0:GD]
        ones_v = scr[GD]
        acc_sh = scr[GD + 1]
        lsem = scr[GD + 2:GD + 2 + GD]
        ssem = scr[GD + 2 + GD:GD + 2 + 2 * GD]
        c = lax.axis_index("c")
        s = lax.axis_index("s")
        base = (c * NS + s) * BPW

        pltpu.sync_copy(zeros_hbm, ones_v)
        zoff = 0
        while zoff < ZR:
            step = min(EB, ZR - zoff)
            pltpu.sync_copy(ones_v.at[pl.ds(0, step)],
                            acc_sh.at[pl.ds(s * ZR + zoff, step)])
            zoff += step
        pltpu.sync_copy(ones_hbm, ones_v)
        plsc.subcore_barrier()

        def fire_load(k, blk):
            off = pl.multiple_of((base + blk) * EB, EB)
            pltpu.async_copy(col_hbm.at[pl.ds(off, EB)], idx[k], lsem[k])

        def wait_load(k, blk):
            off = pl.multiple_of((base + blk) * EB, EB)
            pltpu.make_async_copy(col_hbm.at[pl.ds(off, EB)], idx[k], lsem[k]).wait()

        for k in range(GD):
            fire_load(k, k)

        def body(i, carry):
            for k in range(GD):
                wait_load(k, i * GD + k)
                pltpu.async_copy(ones_v, acc_sh.at[idx[k]], ssem[k], add=True)
            for k in range(GD):
                pltpu.make_async_copy(ones_v, acc_sh.at[idx[k]], ssem[k]).wait()

                @pl.when(i < NGRP - 1)
                def _():
                    fire_load(k, (i + 1) * GD + k)
            return carry

        lax.fori_loop(0, NGRP, body, 0)
        off = pl.multiple_of((base + NGRP * GD) * EB, EB)
        pltpu.sync_copy(col_hbm.at[pl.ds(off, EB)], idx[0])
        pltpu.sync_copy(ones_v, acc_sh.at[idx[0]], add=True)

        plsc.subcore_barrier()
        zoff = 0
        while zoff < ZR:
            step = min(EB, ZR - zoff)
            pltpu.sync_copy(acc_sh.at[pl.ds(s * ZR + zoff, step)],
                            ones_v.at[pl.ds(0, step)])
            pltpu.sync_copy(ones_v.at[pl.ds(0, step)],
                            out_hbm.at[c, pl.ds(s * ZR + zoff, step)])
            zoff += step

    return gather2, scatter_add, count


def _sc_gather2(x, idx2):
    return _sc_build()[0](x, idx2)


def _sc_scatter_add(m, col_s, zeros):
    return _sc_build()[1](m, col_s, zeros)


def _sc_count(col_s, ones, zeros):
    return _sc_build()[2](col_s, ones, zeros)


BF16 = jnp.bfloat16
U32 = jnp.uint32
U16 = jnp.uint16


def _unpack2(p):
    u = lax.bitcast_convert_type(p, U32)
    lo = lax.bitcast_convert_type((u & 0xFFFF).astype(U16), BF16)
    hi = lax.bitcast_convert_type((u >> 16).astype(U16), BF16)
    return jnp.concatenate([lo, hi], axis=1)


def _pack2(x16):
    lo = lax.bitcast_convert_type(x16[:, :DP], U16).astype(U32)
    hi = lax.bitcast_convert_type(x16[:, DP:], U16).astype(U32)
    return lax.bitcast_convert_type(lo | (hi << 16), F32)


def _edge_body(xr, xd, ea, wes, wed, wee, be, wn1x, wn1e, bn1, e_ref, m_ref):
    xr16 = _unpack2(xr[...])
    xd16 = _unpack2(xd[...])
    acc = jnp.dot(xr16, wes[...], preferred_element_type=F32)
    acc += jnp.dot(xd16, wed[...], preferred_element_type=F32)
    acc += jnp.dot(ea[...], wee[...], preferred_element_type=F32)
    e16 = jnp.maximum(acc + be[...], 0.0).astype(BF16)
    e_ref[...] = e16
    m = jnp.dot(xr16, wn1x[...], preferred_element_type=F32)
    m += jnp.dot(e16, wn1e[...], preferred_element_type=F32)
    m_ref[...] = jnp.maximum(m + bn1[...], 0.0)


def _tc_edge(xr, xd, ea, wes, wed, wee, be, wn1x, wn1e, bn1):
    ei = ea.shape[1]
    cst = lambda i: (0, 0)
    row = lambda i: (i, 0)
    return pl.pallas_call(
        _edge_body,
        grid=(GE,),
        in_specs=[
            pl.BlockSpec((TE, DP), row),
            pl.BlockSpec((TE, DP), row),
            pl.BlockSpec((TE, ei), row),
            pl.BlockSpec((D, EO), cst),
            pl.BlockSpec((D, EO), cst),
            pl.BlockSpec((ei, EO), cst),
            pl.BlockSpec((1, EO), cst),
            pl.BlockSpec((D, D), cst),
            pl.BlockSpec((EO, D), cst),
            pl.BlockSpec((1, D), cst),
        ],
        out_specs=[pl.BlockSpec((TE, EO), row), pl.BlockSpec((TE, D), row)],
        out_shape=[jax.ShapeDtypeStruct((E_PAD, EO), BF16),
                   jax.ShapeDtypeStruct((E_PAD, D), F32)],
    )(xr, xd, ea, wes, wed, wee, be, wn1x, wn1e, bn1)


def _node_body(x, p0, p1, invc, b3, wn2x, wn2a, bn2, xn_ref, gs_ref, gc_ref):
    i = pl.program_id(0)
    agg = (p0[0] + p1[0]) * invc[...]
    xn = jnp.dot(_unpack2(x[...]), wn2x[...], preferred_element_type=F32)
    xn += jnp.dot(agg, wn2a[...], preferred_element_type=F32)
    xn = jnp.maximum(xn + bn2[...], 0.0)
    xn_ref[...] = _pack2(xn.astype(BF16))
    seg = b3[0]
    ids = lax.broadcasted_iota(I32, (NG, TN), 0)
    oh = (ids == seg).astype(F32)

    @pl.when(i == 0)
    def _init():
        gs_ref[...] = jnp.zeros_like(gs_ref)
        gc_ref[...] = jnp.zeros_like(gc_ref)

    gs_ref[...] += jnp.dot(oh, xn, preferred_element_type=F32)
    gc_ref[...] += jnp.broadcast_to(jnp.sum(oh, axis=1, keepdims=True), (NG, D))


def _tc_node(x, parts, invc, b3, wn2x, wn2a, bn2):
    cst = lambda i: (0, 0)
    return pl.pallas_call(
        _node_body,
        grid=(GN,),
        in_specs=[
            pl.BlockSpec((TN, DP), lambda i: (i, 0)),
            pl.BlockSpec((1, TN, D), lambda i: (0, i, 0)),
            pl.BlockSpec((1, TN, D), lambda i: (1, i, 0)),
            pl.BlockSpec((TN, D), lambda i: (i, 0)),
            pl.BlockSpec((1, 1, TN), lambda i: (i, 0, 0)),
            pl.BlockSpec((D, D), cst),
            pl.BlockSpec((D, D), cst),
            pl.BlockSpec((1, D), cst),
        ],
        out_specs=[pl.BlockSpec((TN, DP), lambda i: (i, 0)),
                   pl.BlockSpec((NG, D), cst),
                   pl.BlockSpec((NG, D), cst)],
        out_shape=[jax.ShapeDtypeStruct((N, DP), F32),
                   jax.ShapeDtypeStruct((NG, D), F32),
                   jax.ShapeDtypeStruct((NG, D), F32)],
    )(x, parts, parts, invc, b3, wn2x, wn2a, bn2)


def _inv_body(c0, c1, out):
    out[...] = 1.0 / jnp.maximum(c0[0] + c1[0], 1.0)


def _tc_invcnt(cnt_parts):
    return pl.pallas_call(
        _inv_body,
        grid=(GN,),
        in_specs=[pl.BlockSpec((1, TN, D), lambda i: (0, i, 0)),
                  pl.BlockSpec((1, TN, D), lambda i: (1, i, 0))],
        out_specs=pl.BlockSpec((TN, D), lambda i: (i, 0)),
        out_shape=jax.ShapeDtypeStruct((N, D), F32),
    )(cnt_parts, cnt_parts)


def _tc_global(gs, gc, u, wgu, wgm, bg):
    go = wgm.shape[1]

    if u is None:
        def body(gs_r, gc_r, wgm_r, bg_r, out):
            gm = gs_r[...] / jnp.maximum(gc_r[...], 1.0)
            out[...] = jnp.dot(gm, wgm_r[...], preferred_element_type=F32) + bg_r[...]
        args = (gs, gc, wgm, bg)
    else:
        def body(gs_r, gc_r, u_r, wgu_r, wgm_r, bg_r, out):
            gm = gs_r[...] / jnp.maximum(gc_r[...], 1.0)
            acc = jnp.dot(gm, wgm_r[...], preferred_element_type=F32)
            acc += jnp.dot(u_r[...], wgu_r[...], preferred_element_type=F32)
            out[...] = acc + bg_r[...]
        args = (gs, gc, u, wgu, wgm, bg)

    return pl.pallas_call(
        body,
        out_shape=jax.ShapeDtypeStruct((NG, go), F32),
    )(*args)


def _head_body(rg, pg, wr, br, wp, bp, wyr, wyp, by, out):
    a = jnp.dot(rg[...], wr[...], preferred_element_type=F32) + br[...]
    a = jnp.where(a > 0, a, jnp.exp(jnp.minimum(a, 0.0)) - 1.0)
    b = jnp.dot(pg[...], wp[...], preferred_element_type=F32) + bp[...]
    b = jnp.where(b > 0, b, jnp.exp(jnp.minimum(b, 0.0)) - 1.0)
    y = jnp.dot(a, wyr[...], preferred_element_type=F32)
    y += jnp.dot(b, wyp[...], preferred_element_type=F32)
    y = jax.nn.sigmoid(y + by[...]) * 100.0
    out[...] = jnp.broadcast_to(y[:, :1], (NG, D))


def _tc_head(rg, pg, wr, br, wp, bp, wyr, wyp, by):
    return pl.pallas_call(
        _head_body,
        out_shape=jax.ShapeDtypeStruct((NG, D), F32),
    )(rg, pg, wr, br, wp, bp, wyr, wyp, by)


def _meta_layer(x, ea, u, idx2, col_s, invc, b3, zeros, prm):
    (wes, wed, wee, be, wn1x, wn1e, bn1, wn2x, wn2a, bn2, wgu, wgm, bg) = prm
    xr, xd = _sc_gather2(x, idx2)
    e, m = _tc_edge(xr, xd, ea, wes, wed, wee, be, wn1x, wn1e, bn1)
    parts = _sc_scatter_add(m, col_s, zeros)
    xn, gs, gc = _tc_node(x, parts, invc, b3, wn2x, wn2a, bn2)
    un = _tc_global(gs, gc, u, wgu, wgm, bg)
    return xn, e, un


def _split_layer(We, be, Wn1, bn1, Wn2, bn2, Wg, bg, gi, fold_bn=None):
    wes, wed, wee = We[:D], We[D:2 * D], We[2 * D:]
    wn1x, wn1e = Wn1[:D], Wn1[D:]
    wn2x, wn2a = Wn2[:D], Wn2[D:]
    if fold_bn is not None:
        nsc, nsh, esc, esh = fold_bn
        be = be + nsh @ wes + nsh @ wed + esh @ wee
        wes = nsc[:, None] * wes
        wed = nsc[:, None] * wed
        wee = esc[:, None] * wee
        bn1 = bn1 + nsh @ wn1x
        wn1x = nsc[:, None] * wn1x
        bn2 = bn2 + nsh @ wn2x
        wn2x = nsc[:, None] * wn2x
    wgu, wgm = (Wg[:gi], Wg[gi:]) if gi > 0 else (None, Wg)
    b16 = jnp.bfloat16
    return (wes.astype(b16), wed.astype(b16), wee.astype(b16), be[None, :],
            wn1x.astype(b16), wn1e.astype(b16), bn1[None, :],
            wn2x.astype(b16), wn2a, bn2[None, :], wgu, wgm, bg[None, :])


def kernel(rx, re, rg, px, pe, rc, pc, rb, pb, bnn_g, bnn_b, bnn_m, bnn_v, bne_g, bne_b, bne_m, bne_v, We_r1, be_r1, Wn1_r1, bn1_r1, Wn2_r1, bn2_r1, Wg_r1, bg_r1, We_r2, be_r2, Wn1_r2, bn1_r2, Wn2_r2, bn2_r2, Wg_r2, bg_r2, We_r3, be_r3, Wn1_r3, bn1_r3, Wn2_r3, bn2_r3, Wg_r3, bg_r3, We_r4, be_r4, Wn1_r4, bn1_r4, Wn2_r4, bn2_r4, Wg_r4, bg_r4, We_r5, be_r5, Wn1_r5, bn1_r5, Wn2_r5, bn2_r5, Wg_r5, bg_r5, We_p1, be_p1, Wn1_p1, bn1_p1, Wn2_p1, bn2_p1, Wg_p1, bg_p1, We_p2, be_p2, Wn1_p2, bn1_p2, Wn2_p2, bn2_p2, Wg_p2, bg_p2, We_p3, be_p3, Wn1_p3, bn1_p3, Wn2_p3, bn2_p3, Wg_p3, bg_p3, W_rlin, b_rlin, W_plin, b_plin, W_y, b_y):
    pad = E_PAD - E
    zpad = jnp.zeros((pad,), I32)
    npad = jnp.full((pad,), N, I32)
    rrow = jnp.concatenate([rc[0].astype(I32), zpad])
    rcol_g = jnp.concatenate([rc[1].astype(I32), zpad])
    rcol_s = jnp.concatenate([rc[1].astype(I32), npad])
    prow = jnp.concatenate([pc[0].astype(I32), zpad])
    pcol_g = jnp.concatenate([pc[1].astype(I32), zpad])
    pcol_s = jnp.concatenate([pc[1].astype(I32), npad])
    ridx2 = jnp.stack([rrow.reshape(NBLK, EB), rcol_g.reshape(NBLK, EB)], axis=1)
    pidx2 = jnp.stack([prow.reshape(NBLK, EB), pcol_g.reshape(NBLK, EB)], axis=1)
    rb3 = rb.astype(I32).reshape(GN, 1, TN)
    pb3 = pb.astype(I32).reshape(GN, 1, TN)

    nsc = bnn_g / jnp.sqrt(bnn_v + EPS)
    nsh = bnn_b - bnn_m * nsc
    esc = bne_g / jnp.sqrt(bne_v + EPS)
    esh = bne_b - bne_m * esc
    fold = (nsc, nsh, esc, esh)

    re_p = jnp.concatenate([re, jnp.zeros((pad, re.shape[1]), F32)]).astype(jnp.bfloat16)
    pe_p = jnp.concatenate([pe, jnp.zeros((pad, pe.shape[1]), F32)]).astype(jnp.bfloat16)

    zeros = jnp.zeros((EB, D), F32)
    ones = jnp.ones((EB, D), F32)

    def pack_host(x):
        x16 = x.astype(jnp.bfloat16)
        lo = lax.bitcast_convert_type(x16[:, :DP], jnp.uint16).astype(jnp.uint32)
        hi = lax.bitcast_convert_type(x16[:, DP:], jnp.uint16).astype(jnp.uint32)
        return lax.bitcast_convert_type(lo | (hi << 16), F32)

    rx_p = pack_host(rx)
    px_p = pack_host(px)

    r_layers = [
        _split_layer(We_r1, be_r1, Wn1_r1, bn1_r1, Wn2_r1, bn2_r1, Wg_r1, bg_r1, 32, fold),
        _split_layer(We_r2, be_r2, Wn1_r2, bn1_r2, Wn2_r2, bn2_r2, Wg_r2, bg_r2, 128),
        _split_layer(We_r3, be_r3, Wn1_r3, bn1_r3, Wn2_r3, bn2_r3, Wg_r3, bg_r3, 128),
        _split_layer(We_r4, be_r4, Wn1_r4, bn1_r4, Wn2_r4, bn2_r4, Wg_r4, bg_r4, 128),
        _split_layer(We_r5, be_r5, Wn1_r5, bn1_r5, Wn2_r5, bn2_r5, Wg_r5, bg_r5, 128),
    ]
    p_layers = [
        _split_layer(We_p1, be_p1, Wn1_p1, bn1_p1, Wn2_p1, bn2_p1, Wg_p1, bg_p1, 0, fold),
        _split_layer(We_p2, be_p2, Wn1_p2, bn1_p2, Wn2_p2, bn2_p2, Wg_p2, bg_p2, 128),
        _split_layer(We_p3, be_p3, Wn1_p3, bn1_p3, Wn2_p3, bn2_p3, Wg_p3, bg_p3, 128),
    ]

    r_invc = _tc_invcnt(_sc_count(rcol_s, ones, zeros))
    p_invc = _tc_invcnt(_sc_count(pcol_s, ones, zeros))

    rst = (rx_p, re_p, rg)
    pst = (px_p, pe_p, None)
    for i in range(5):
        rst = _meta_layer(*rst, ridx2, rcol_s, r_invc, rb3, zeros, r_layers[i])
        if i < 3:
            pst = _meta_layer(*pst, pidx2, pcol_s, p_invc, pb3, zeros, p_layers[i])
    rg_out = rst[2]
    pg_out = pst[2]

    y = _tc_head(rg_out, pg_out,
                 W_rlin, b_rlin[None, :], W_plin, b_plin[None, :],
                 W_y[:256], W_y[256:], b_y[None, :])
    return y[:, :1]


# speedup vs baseline: 1.3097x; 1.2712x over previous
import functools

import jax
import jax.numpy as jnp
from jax import lax
from jax.experimental import pallas as pl
from jax.experimental.pallas import tpu as pltpu
from jax.experimental.pallas import tpu_sc as plsc

F32 = jnp.float32
I32 = jnp.int32

N = 10000
E = 160000
D = 128
EO = 512
NG = 16
EPS = 1e-5

NC = 2
NS = 16
NW = NC * NS
EB = 128
E_PAD = 163840
NBLK = E_PAD // EB
BPW = E_PAD // (NW * EB)
N_PAD = 10112
ZR = N_PAD // NS

DP = 64

TE = 640
GE = E_PAD // TE
TN = 1000
GN = N // TN


@functools.cache
def _sc_build():
    mesh = plsc.VectorSubcoreMesh(core_axis_name="c", subcore_axis_name="s")
    GD = 3
    NGRP = BPW // GD
    GDG = 5
    NGRPG = BPW // GDG

    @functools.partial(
        pl.kernel,
        out_type=(jax.ShapeDtypeStruct((E_PAD, DP), F32),
                  jax.ShapeDtypeStruct((E_PAD, DP), F32)),
        mesh=mesh,
        compiler_params=pltpu.CompilerParams(use_tc_tiling_on_sc=False),
        scratch_types=(
            [pltpu.VMEM((2, EB), I32) for _ in range(GDG)]
            + [pltpu.VMEM((2 * EB, DP), F32) for _ in range(GDG)]
            + [pltpu.VMEM_SHARED((N_PAD, DP), F32)]
            + [pltpu.SemaphoreType.DMA for _ in range(2 * GDG)]
        ),
    )
    def gather2(x_hbm, idx2_hbm, outr_hbm, outc_hbm, *scr):
        ib = scr[0:GDG]
        bd = scr[GDG:2 * GDG]
        xs_sh = scr[2 * GDG]
        sems = scr[2 * GDG + 1:4 * GDG + 1]
        wid = lax.axis_index("c") * NS + lax.axis_index("s")
        base = wid * BPW
        s = lax.axis_index("s")
        pltpu.sync_copy(x_hbm.at[pl.ds(s * ZR, ZR)], xs_sh.at[pl.ds(s * ZR, ZR)])
        plsc.subcore_barrier()

        def do_group(first_blk, nk):
            for k in range(nk):
                g = base + first_blk + k
                pltpu.sync_copy(idx2_hbm.at[g], ib[k])
                pltpu.async_copy(xs_sh.at[ib[k].at[0]], bd[k].at[pl.ds(0, EB)],
                                 sems[2 * k])
                pltpu.async_copy(xs_sh.at[ib[k].at[1]], bd[k].at[pl.ds(EB, EB)],
                                 sems[2 * k + 1])
            for k in range(nk):
                g = base + first_blk + k
                off = pl.multiple_of(g * EB, EB)
                pltpu.make_async_copy(xs_sh.at[ib[k].at[0]],
                                      bd[k].at[pl.ds(0, EB)], sems[2 * k]).wait()
                pltpu.make_async_copy(xs_sh.at[ib[k].at[1]],
                                      bd[k].at[pl.ds(EB, EB)], sems[2 * k + 1]).wait()
                pltpu.sync_copy(bd[k].at[pl.ds(0, EB)], outr_hbm.at[pl.ds(off, EB)])
                pltpu.sync_copy(bd[k].at[pl.ds(EB, EB)], outc_hbm.at[pl.ds(off, EB)])

        def body(i, carry):
            do_group(i * GDG, GDG)
            return carry

        lax.fori_loop(0, NGRPG, body, 0)
        if BPW % GDG:
            do_group(NGRPG * GDG, BPW % GDG)

    @functools.partial(
        pl.kernel,
        out_type=jax.ShapeDtypeStruct((NC, N_PAD, D), F32),
        mesh=mesh,
        scratch_types=(
            [pltpu.VMEM((EB,), I32) for _ in range(GD)]
            + [pltpu.VMEM((EB, D), F32) for _ in range(GD)]
            + [pltpu.VMEM_SHARED((N_PAD, D), F32)]
            + [pltpu.SemaphoreType.DMA for _ in range(2 * GD)]
        ),
    )
    def scatter_add(m_hbm, col_hbm, zeros_hbm, out_hbm, *scr):
        idx = scr[0:GD]
        mb = scr[GD:2 * GD]
        acc_sh = scr[2 * GD]
        lsem = scr[2 * GD + 1:2 * GD + 1 + GD]
        ssem = scr[2 * GD + 1 + GD:2 * GD + 1 + 2 * GD]
        c = lax.axis_index("c")
        s = lax.axis_index("s")
        base = (c * NS + s) * BPW

        pltpu.sync_copy(zeros_hbm, mb[0])
        zoff = 0
        while zoff < ZR:
            step = min(EB, ZR - zoff)
            pltpu.sync_copy(mb[0].at[pl.ds(0, step)],
                            acc_sh.at[pl.ds(s * ZR + zoff, step)])
            zoff += step
        plsc.subcore_barrier()

        def fire_loads(k, blk):
            off = pl.multiple_of((base + blk) * EB, EB)
            pltpu.async_copy(col_hbm.at[pl.ds(off, EB)], idx[k], lsem[k])
            pltpu.async_copy(m_hbm.at[pl.ds(off, EB)], mb[k], lsem[k])

        def wait_loads(k, blk):
            off = pl.multiple_of((base + blk) * EB, EB)
            pltpu.make_async_copy(col_hbm.at[pl.ds(off, EB)], idx[k], lsem[k]).wait()
            pltpu.make_async_copy(m_hbm.at[pl.ds(off, EB)], mb[k], lsem[k]).wait()

        for k in range(GD):
            fire_loads(k, k)

        def body(i, carry):
            for k in range(GD):
                wait_loads(k, i * GD + k)
                pltpu.async_copy(mb[k], acc_sh.at[idx[k]], ssem[k], add=True)
            for k in range(GD):
                pltpu.make_async_copy(mb[k], acc_sh.at[idx[k]], ssem[k]).wait()

                @pl.when(i < NGRP - 1)
                def _():
                    fire_loads(k, (i + 1) * GD + k)
            return carry

        lax.fori_loop(0, NGRP, body, 0)
        off = pl.multiple_of((base + NGRP * GD) * EB, EB)
        pltpu.sync_copy(col_hbm.at[pl.ds(off, EB)], idx[0])
        pltpu.sync_copy(m_hbm.at[pl.ds(off, EB)], mb[0])
        pltpu.sync_copy(mb[0], acc_sh.at[idx[0]], add=True)

        plsc.subcore_barrier()
        zoff = 0
        while zoff < ZR:
            step = min(EB, ZR - zoff)
            pltpu.sync_copy(acc_sh.at[pl.ds(s * ZR + zoff, step)],
                            mb[0].at[pl.ds(0, step)])
            pltpu.sync_copy(mb[0].at[pl.ds(0, step)],
                            out_hbm.at[c, pl.ds(s * ZR + zoff, step)])
            zoff += step

    @functools.partial(
        pl.kernel,
        out_type=jax.ShapeDtypeStruct((NC, N_PAD, D), F32),
        mesh=mesh,
        scratch_types=(
            [pltpu.VMEM((EB,), I32) for _ in range(GD)]
            + [pltpu.VMEM((EB, D), F32)]
            + [pltpu.VMEM_SHARED((N_PAD, D), F32)]
            + [pltpu.SemaphoreType.DMA for _ in range(2 * GD)]
        ),
    )
    def count(col_hbm, ones_hbm, zeros_hbm, out_hbm, *scr):
        idx = scr[0:GD]
        ones_v = scr[GD]
        acc_sh = scr[GD + 1]
        lsem = scr[GD + 2:GD + 2 + GD]
        ssem = scr[GD + 2 + GD:GD + 2 + 2 * GD]
        c = lax.axis_index("c")
        s = lax.axis_index("s")
        base = (c * NS + s) * BPW

        pltpu.sync_copy(zeros_hbm, ones_v)
        zoff = 0
        while zoff < ZR:
            step = min(EB, ZR - zoff)
            pltpu.sync_copy(ones_v.at[pl.ds(0, step)],
                            acc_sh.at[pl.ds(s * ZR + zoff, step)])
            zoff += step
        pltpu.sync_copy(ones_hbm, ones_v)
        plsc.subcore_barrier()

        def fire_load(k, blk):
            off = pl.multiple_of((base + blk) * EB, EB)
            pltpu.async_copy(col_hbm.at[pl.ds(off, EB)], idx[k], lsem[k])

        def wait_load(k, blk):
            off = pl.multiple_of((base + blk) * EB, EB)
            pltpu.make_async_copy(col_hbm.at[pl.ds(off, EB)], idx[k], lsem[k]).wait()

        for k in range(GD):
            fire_load(k, k)

        def body(i, carry):
            for k in range(GD):
                wait_load(k, i * GD + k)
                pltpu.async_copy(ones_v, acc_sh.at[idx[k]], ssem[k], add=True)
            for k in range(GD):
                pltpu.make_async_copy(ones_v, acc_sh.at[idx[k]], ssem[k]).wait()

                @pl.when(i < NGRP - 1)
                def _():
                    fire_load(k, (i + 1) * GD + k)
            return carry

        lax.fori_loop(0, NGRP, body, 0)
        off = pl.multiple_of((base + NGRP * GD) * EB, EB)
        pltpu.sync_copy(col_hbm.at[pl.ds(off, EB)], idx[0])
        pltpu.sync_copy(ones_v, acc_sh.at[idx[0]], add=True)

        plsc.subcore_barrier()
        zoff = 0
        while zoff < ZR:
            step = min(EB, ZR - zoff)
            pltpu.sync_copy(acc_sh.at[pl.ds(s * ZR + zoff, step)],
                            ones_v.at[pl.ds(0, step)])
            pltpu.sync_copy(ones_v.at[pl.ds(0, step)],
                            out_hbm.at[c, pl.ds(s * ZR + zoff, step)])
            zoff += step

    return gather2, scatter_add, count


def _sc_gather2(x, idx2):
    return _sc_build()[0](x, idx2)


def _sc_scatter_add(m, col_s, zeros):
    return _sc_build()[1](m, col_s, zeros)


def _sc_count(col_s, ones, zeros):
    return _sc_build()[2](col_s, ones, zeros)


BF16 = jnp.bfloat16
U32 = jnp.uint32
U16 = jnp.uint16


def _unpack2(p):
    u = lax.bitcast_convert_type(p, U32)
    lo = lax.bitcast_convert_type((u & 0xFFFF).astype(U16), BF16)
    hi = lax.bitcast_convert_type((u >> 16).astype(U16), BF16)
    return jnp.concatenate([lo, hi], axis=1)


def _pack2(x16):
    lo = lax.bitcast_convert_type(x16[:, :DP], U16).astype(U32)
    hi = lax.bitcast_convert_type(x16[:, DP:], U16).astype(U32)
    return lax.bitcast_convert_type(lo | (hi << 16), F32)


def _edge_body(xr, xd, ea, wes, wed, wee, be, wn1x, wn1e, bn1, e_ref, m_ref):
    xr16 = _unpack2(xr[...])
    xd16 = _unpack2(xd[...])
    acc = jnp.dot(xr16, wes[...], preferred_element_type=F32)
    acc += jnp.dot(xd16, wed[...], preferred_element_type=F32)
    acc += jnp.dot(ea[...], wee[...], preferred_element_type=F32)
    e16 = jnp.maximum(acc + be[...], 0.0).astype(BF16)
    e_ref[...] = e16
    m = jnp.dot(xr16, wn1x[...], preferred_element_type=F32)
    m += jnp.dot(e16, wn1e[...], preferred_element_type=F32)
    m_ref[...] = jnp.maximum(m + bn1[...], 0.0)


def _tc_edge(xr, xd, ea, wes, wed, wee, be, wn1x, wn1e, bn1):
    ei = ea.shape[1]
    cst = lambda i: (0, 0)
    row = lambda i: (i, 0)
    return pl.pallas_call(
        _edge_body,
        grid=(GE,),
        in_specs=[
            pl.BlockSpec((TE, DP), row),
            pl.BlockSpec((TE, DP), row),
            pl.BlockSpec((TE, ei), row),
            pl.BlockSpec((D, EO), cst),
            pl.BlockSpec((D, EO), cst),
            pl.BlockSpec((ei, EO), cst),
            pl.BlockSpec((1, EO), cst),
            pl.BlockSpec((D, D), cst),
            pl.BlockSpec((EO, D), cst),
            pl.BlockSpec((1, D), cst),
        ],
        out_specs=[pl.BlockSpec((TE, EO), row), pl.BlockSpec((TE, D), row)],
        out_shape=[jax.ShapeDtypeStruct((E_PAD, EO), BF16),
                   jax.ShapeDtypeStruct((E_PAD, D), F32)],
    )(xr, xd, ea, wes, wed, wee, be, wn1x, wn1e, bn1)


def _node_body(x, p0, p1, invc, b3, wn2x, wn2a, bn2, xn_ref, gs_ref, gc_ref):
    i = pl.program_id(0)
    agg = (p0[0] + p1[0]) * invc[...]
    xn = jnp.dot(_unpack2(x[...]), wn2x[...], preferred_element_type=F32)
    xn += jnp.dot(agg, wn2a[...], preferred_element_type=F32)
    xn = jnp.maximum(xn + bn2[...], 0.0)
    xn_ref[...] = _pack2(xn.astype(BF16))
    seg = b3[0]
    ids = lax.broadcasted_iota(I32, (NG, TN), 0)
    oh = (ids == seg).astype(F32)

    @pl.when(i == 0)
    def _init():
        gs_ref[...] = jnp.zeros_like(gs_ref)
        gc_ref[...] = jnp.zeros_like(gc_ref)

    gs_ref[...] += jnp.dot(oh, xn, preferred_element_type=F32)
    gc_ref[...] += jnp.broadcast_to(jnp.sum(oh, axis=1, keepdims=True), (NG, D))


def _tc_node(x, parts, invc, b3, wn2x, wn2a, bn2):
    cst = lambda i: (0, 0)
    return pl.pallas_call(
        _node_body,
        grid=(GN,),
        in_specs=[
            pl.BlockSpec((TN, DP), lambda i: (i, 0)),
            pl.BlockSpec((1, TN, D), lambda i: (0, i, 0)),
            pl.BlockSpec((1, TN, D), lambda i: (1, i, 0)),
            pl.BlockSpec((TN, D), lambda i: (i, 0)),
            pl.BlockSpec((1, 1, TN), lambda i: (i, 0, 0)),
            pl.BlockSpec((D, D), cst),
            pl.BlockSpec((D, D), cst),
            pl.BlockSpec((1, D), cst),
        ],
        out_specs=[pl.BlockSpec((TN, DP), lambda i: (i, 0)),
                   pl.BlockSpec((NG, D), cst),
                   pl.BlockSpec((NG, D), cst)],
        out_shape=[jax.ShapeDtypeStruct((N_PAD, DP), F32),
                   jax.ShapeDtypeStruct((NG, D), F32),
                   jax.ShapeDtypeStruct((NG, D), F32)],
    )(x, parts, parts, invc, b3, wn2x, wn2a, bn2)


def _inv_body(c0, c1, out):
    out[...] = 1.0 / jnp.maximum(c0[0] + c1[0], 1.0)


def _tc_invcnt(cnt_parts):
    return pl.pallas_call(
        _inv_body,
        grid=(GN,),
        in_specs=[pl.BlockSpec((1, TN, D), lambda i: (0, i, 0)),
                  pl.BlockSpec((1, TN, D), lambda i: (1, i, 0))],
        out_specs=pl.BlockSpec((TN, D), lambda i: (i, 0)),
        out_shape=jax.ShapeDtypeStruct((N, D), F32),
    )(cnt_parts, cnt_parts)


def _tc_global(gs, gc, u, wgu, wgm, bg):
    go = wgm.shape[1]

    if u is None:
        def body(gs_r, gc_r, wgm_r, bg_r, out):
            gm = gs_r[...] / jnp.maximum(gc_r[...], 1.0)
            out[...] = jnp.dot(gm, wgm_r[...], preferred_element_type=F32) + bg_r[...]
        args = (gs, gc, wgm, bg)
    else:
        def body(gs_r, gc_r, u_r, wgu_r, wgm_r, bg_r, out):
            gm = gs_r[...] / jnp.maximum(gc_r[...], 1.0)
            acc = jnp.dot(gm, wgm_r[...], preferred_element_type=F32)
            acc += jnp.dot(u_r[...], wgu_r[...], preferred_element_type=F32)
            out[...] = acc + bg_r[...]
        args = (gs, gc, u, wgu, wgm, bg)

    return pl.pallas_call(
        body,
        out_shape=jax.ShapeDtypeStruct((NG, go), F32),
    )(*args)


def _head_body(rg, pg, wr, br, wp, bp, wyr, wyp, by, out):
    a = jnp.dot(rg[...], wr[...], preferred_element_type=F32) + br[...]
    a = jnp.where(a > 0, a, jnp.exp(jnp.minimum(a, 0.0)) - 1.0)
    b = jnp.dot(pg[...], wp[...], preferred_element_type=F32) + bp[...]
    b = jnp.where(b > 0, b, jnp.exp(jnp.minimum(b, 0.0)) - 1.0)
    y = jnp.dot(a, wyr[...], preferred_element_type=F32)
    y += jnp.dot(b, wyp[...], preferred_element_type=F32)
    y = jax.nn.sigmoid(y + by[...]) * 100.0
    out[...] = jnp.broadcast_to(y[:, :1], (NG, D))


def _tc_head(rg, pg, wr, br, wp, bp, wyr, wyp, by):
    return pl.pallas_call(
        _head_body,
        out_shape=jax.ShapeDtypeStruct((NG, D), F32),
    )(rg, pg, wr, br, wp, bp, wyr, wyp, by)


def _meta_layer(x, ea, u, idx2, col_s, invc, b3, zeros, prm):
    (wes, wed, wee, be, wn1x, wn1e, bn1, wn2x, wn2a, bn2, wgu, wgm, bg) = prm
    xr, xd = _sc_gather2(x, idx2)
    e, m = _tc_edge(xr, xd, ea, wes, wed, wee, be, wn1x, wn1e, bn1)
    parts = _sc_scatter_add(m, col_s, zeros)
    xn, gs, gc = _tc_node(x, parts, invc, b3, wn2x, wn2a, bn2)
    un = _tc_global(gs, gc, u, wgu, wgm, bg)
    return xn, e, un


def _split_layer(We, be, Wn1, bn1, Wn2, bn2, Wg, bg, gi, fold_bn=None):
    wes, wed, wee = We[:D], We[D:2 * D], We[2 * D:]
    wn1x, wn1e = Wn1[:D], Wn1[D:]
    wn2x, wn2a = Wn2[:D], Wn2[D:]
    if fold_bn is not None:
        nsc, nsh, esc, esh = fold_bn
        be = be + nsh @ wes + nsh @ wed + esh @ wee
        wes = nsc[:, None] * wes
        wed = nsc[:, None] * wed
        wee = esc[:, None] * wee
        bn1 = bn1 + nsh @ wn1x
        wn1x = nsc[:, None] * wn1x
        bn2 = bn2 + nsh @ wn2x
        wn2x = nsc[:, None] * wn2x
    wgu, wgm = (Wg[:gi], Wg[gi:]) if gi > 0 else (None, Wg)
    b16 = jnp.bfloat16
    return (wes.astype(b16), wed.astype(b16), wee.astype(b16), be[None, :],
            wn1x.astype(b16), wn1e.astype(b16), bn1[None, :],
            wn2x.astype(b16), wn2a, bn2[None, :], wgu, wgm, bg[None, :])


def kernel(rx, re, rg, px, pe, rc, pc, rb, pb, bnn_g, bnn_b, bnn_m, bnn_v, bne_g, bne_b, bne_m, bne_v, We_r1, be_r1, Wn1_r1, bn1_r1, Wn2_r1, bn2_r1, Wg_r1, bg_r1, We_r2, be_r2, Wn1_r2, bn1_r2, Wn2_r2, bn2_r2, Wg_r2, bg_r2, We_r3, be_r3, Wn1_r3, bn1_r3, Wn2_r3, bn2_r3, Wg_r3, bg_r3, We_r4, be_r4, Wn1_r4, bn1_r4, Wn2_r4, bn2_r4, Wg_r4, bg_r4, We_r5, be_r5, Wn1_r5, bn1_r5, Wn2_r5, bn2_r5, Wg_r5, bg_r5, We_p1, be_p1, Wn1_p1, bn1_p1, Wn2_p1, bn2_p1, Wg_p1, bg_p1, We_p2, be_p2, Wn1_p2, bn1_p2, Wn2_p2, bn2_p2, Wg_p2, bg_p2, We_p3, be_p3, Wn1_p3, bn1_p3, Wn2_p3, bn2_p3, Wg_p3, bg_p3, W_rlin, b_rlin, W_plin, b_plin, W_y, b_y):
    pad = E_PAD - E
    zpad = jnp.zeros((pad,), I32)
    npad = jnp.full((pad,), N, I32)
    rrow = jnp.concatenate([rc[0].astype(I32), zpad])
    rcol_g = jnp.concatenate([rc[1].astype(I32), zpad])
    rcol_s = jnp.concatenate([rc[1].astype(I32), npad])
    prow = jnp.concatenate([pc[0].astype(I32), zpad])
    pcol_g = jnp.concatenate([pc[1].astype(I32), zpad])
    pcol_s = jnp.concatenate([pc[1].astype(I32), npad])
    ridx2 = jnp.stack([rrow.reshape(NBLK, EB), rcol_g.reshape(NBLK, EB)], axis=1)
    pidx2 = jnp.stack([prow.reshape(NBLK, EB), pcol_g.reshape(NBLK, EB)], axis=1)
    rb3 = rb.astype(I32).reshape(GN, 1, TN)
    pb3 = pb.astype(I32).reshape(GN, 1, TN)

    nsc = bnn_g / jnp.sqrt(bnn_v + EPS)
    nsh = bnn_b - bnn_m * nsc
    esc = bne_g / jnp.sqrt(bne_v + EPS)
    esh = bne_b - bne_m * esc
    fold = (nsc, nsh, esc, esh)

    re_p = jnp.concatenate([re, jnp.zeros((pad, re.shape[1]), F32)]).astype(jnp.bfloat16)
    pe_p = jnp.concatenate([pe, jnp.zeros((pad, pe.shape[1]), F32)]).astype(jnp.bfloat16)

    zeros = jnp.zeros((EB, D), F32)
    ones = jnp.ones((EB, D), F32)

    def pack_host(x):
        x16 = x.astype(jnp.bfloat16)
        lo = lax.bitcast_convert_type(x16[:, :DP], jnp.uint16).astype(jnp.uint32)
        hi = lax.bitcast_convert_type(x16[:, DP:], jnp.uint16).astype(jnp.uint32)
        return lax.bitcast_convert_type(lo | (hi << 16), F32)

    rx_p = jnp.pad(pack_host(rx), ((0, N_PAD - N), (0, 0)))
    px_p = jnp.pad(pack_host(px), ((0, N_PAD - N), (0, 0)))

    r_layers = [
        _split_layer(We_r1, be_r1, Wn1_r1, bn1_r1, Wn2_r1, bn2_r1, Wg_r1, bg_r1, 32, fold),
        _split_layer(We_r2, be_r2, Wn1_r2, bn1_r2, Wn2_r2, bn2_r2, Wg_r2, bg_r2, 128),
        _split_layer(We_r3, be_r3, Wn1_r3, bn1_r3, Wn2_r3, bn2_r3, Wg_r3, bg_r3, 128),
        _split_layer(We_r4, be_r4, Wn1_r4, bn1_r4, Wn2_r4, bn2_r4, Wg_r4, bg_r4, 128),
        _split_layer(We_r5, be_r5, Wn1_r5, bn1_r5, Wn2_r5, bn2_r5, Wg_r5, bg_r5, 128),
    ]
    p_layers = [
        _split_layer(We_p1, be_p1, Wn1_p1, bn1_p1, Wn2_p1, bn2_p1, Wg_p1, bg_p1, 0, fold),
        _split_layer(We_p2, be_p2, Wn1_p2, bn1_p2, Wn2_p2, bn2_p2, Wg_p2, bg_p2, 128),
        _split_layer(We_p3, be_p3, Wn1_p3, bn1_p3, Wn2_p3, bn2_p3, Wg_p3, bg_p3, 128),
    ]

    r_invc = _tc_invcnt(_sc_count(rcol_s, ones, zeros))
    p_invc = _tc_invcnt(_sc_count(pcol_s, ones, zeros))

    rst = (rx_p, re_p, rg)
    pst = (px_p, pe_p, None)
    for i in range(5):
        rst = _meta_layer(*rst, ridx2, rcol_s, r_invc, rb3, zeros, r_layers[i])
        if i < 3:
            pst = _meta_layer(*pst, pidx2, pcol_s, p_invc, pb3, zeros, p_layers[i])
    rg_out = rst[2]
    pg_out = pst[2]

    y = _tc_head(rg_out, pg_out,
                 W_rlin, b_rlin[None, :], W_plin, b_plin[None, :],
                 W_y[:256], W_y[256:], b_y[None, :])
    return y[:, :1]


# speedup vs baseline: 1.5240x; 1.1636x over previous
import functools

import jax
import jax.numpy as jnp
from jax import lax
from jax.experimental import pallas as pl
from jax.experimental.pallas import tpu as pltpu
from jax.experimental.pallas import tpu_sc as plsc

F32 = jnp.float32
I32 = jnp.int32

N = 10000
E = 160000
D = 128
EO = 512
NG = 16
EPS = 1e-5

NC = 2
NS = 16
NW = NC * NS
EB = 128
E_PAD = 163840
NBLK = E_PAD // EB
BPW = E_PAD // (NW * EB)
N_PAD = 10112
ZR = N_PAD // NS

DP = 64

TE = 1280
GE = E_PAD // TE
TN = 1000
GN = N // TN


@functools.cache
def _sc_build():
    mesh = plsc.VectorSubcoreMesh(core_axis_name="c", subcore_axis_name="s")
    GD = 3
    NGRP = BPW // GD
    GDG = 5
    NGRPG = BPW // GDG

    @functools.partial(
        pl.kernel,
        out_type=(jax.ShapeDtypeStruct((E_PAD, DP), F32),
                  jax.ShapeDtypeStruct((E_PAD, DP), F32)),
        mesh=mesh,
        compiler_params=pltpu.CompilerParams(use_tc_tiling_on_sc=False),
        scratch_types=(
            [pltpu.VMEM((2, EB), I32) for _ in range(GDG)]
            + [pltpu.VMEM((2 * EB, DP), F32) for _ in range(GDG)]
            + [pltpu.VMEM_SHARED((N_PAD, DP), F32)]
            + [pltpu.SemaphoreType.DMA for _ in range(2 * GDG)]
        ),
    )
    def gather2(x_hbm, idx2_hbm, outr_hbm, outc_hbm, *scr):
        ib = scr[0:GDG]
        bd = scr[GDG:2 * GDG]
        xs_sh = scr[2 * GDG]
        sems = scr[2 * GDG + 1:4 * GDG + 1]
        wid = lax.axis_index("c") * NS + lax.axis_index("s")
        base = wid * BPW
        s = lax.axis_index("s")
        pltpu.sync_copy(x_hbm.at[pl.ds(s * ZR, ZR)], xs_sh.at[pl.ds(s * ZR, ZR)])
        plsc.subcore_barrier()

        def do_group(first_blk, nk):
            for k in range(nk):
                g = base + first_blk + k
                pltpu.sync_copy(idx2_hbm.at[g], ib[k])
                pltpu.async_copy(xs_sh.at[ib[k].at[0]], bd[k].at[pl.ds(0, EB)],
                                 sems[2 * k])
                pltpu.async_copy(xs_sh.at[ib[k].at[1]], bd[k].at[pl.ds(EB, EB)],
                                 sems[2 * k + 1])
            for k in range(nk):
                g = base + first_blk + k
                off = pl.multiple_of(g * EB, EB)
                pltpu.make_async_copy(xs_sh.at[ib[k].at[0]],
                                      bd[k].at[pl.ds(0, EB)], sems[2 * k]).wait()
                pltpu.make_async_copy(xs_sh.at[ib[k].at[1]],
                                      bd[k].at[pl.ds(EB, EB)], sems[2 * k + 1]).wait()
                pltpu.sync_copy(bd[k].at[pl.ds(0, EB)], outr_hbm.at[pl.ds(off, EB)])
                pltpu.sync_copy(bd[k].at[pl.ds(EB, EB)], outc_hbm.at[pl.ds(off, EB)])

        def body(i, carry):
            do_group(i * GDG, GDG)
            return carry

        lax.fori_loop(0, NGRPG, body, 0)
        if BPW % GDG:
            do_group(NGRPG * GDG, BPW % GDG)

    @functools.partial(
        pl.kernel,
        out_type=jax.ShapeDtypeStruct((NC, N_PAD, D), F32),
        mesh=mesh,
        scratch_types=(
            [pltpu.VMEM((EB,), I32) for _ in range(GD)]
            + [pltpu.VMEM((EB, D), F32) for _ in range(GD)]
            + [pltpu.VMEM_SHARED((N_PAD, D), F32)]
            + [pltpu.SemaphoreType.DMA for _ in range(2 * GD)]
        ),
    )
    def scatter_add(m_hbm, col_hbm, zeros_hbm, out_hbm, *scr):
        idx = scr[0:GD]
        mb = scr[GD:2 * GD]
        acc_sh = scr[2 * GD]
        lsem = scr[2 * GD + 1:2 * GD + 1 + GD]
        ssem = scr[2 * GD + 1 + GD:2 * GD + 1 + 2 * GD]
        c = lax.axis_index("c")
        s = lax.axis_index("s")
        base = (c * NS + s) * BPW

        pltpu.sync_copy(zeros_hbm, mb[0])
        zoff = 0
        while zoff < ZR:
            step = min(EB, ZR - zoff)
            pltpu.sync_copy(mb[0].at[pl.ds(0, step)],
                            acc_sh.at[pl.ds(s * ZR + zoff, step)])
            zoff += step
        plsc.subcore_barrier()

        def fire_loads(k, blk):
            off = pl.multiple_of((base + blk) * EB, EB)
            pltpu.async_copy(col_hbm.at[pl.ds(off, EB)], idx[k], lsem[k])
            pltpu.async_copy(m_hbm.at[pl.ds(off, EB)], mb[k], lsem[k])

        def wait_loads(k, blk):
            off = pl.multiple_of((base + blk) * EB, EB)
            pltpu.make_async_copy(col_hbm.at[pl.ds(off, EB)], idx[k], lsem[k]).wait()
            pltpu.make_async_copy(m_hbm.at[pl.ds(off, EB)], mb[k], lsem[k]).wait()

        for k in range(GD):
            fire_loads(k, k)

        def body(i, carry):
            for k in range(GD):
                wait_loads(k, i * GD + k)
                pltpu.async_copy(mb[k], acc_sh.at[idx[k]], ssem[k], add=True)
            for k in range(GD):
                pltpu.make_async_copy(mb[k], acc_sh.at[idx[k]], ssem[k]).wait()

                @pl.when(i < NGRP - 1)
                def _():
                    fire_loads(k, (i + 1) * GD + k)
            return carry

        lax.fori_loop(0, NGRP, body, 0)
        off = pl.multiple_of((base + NGRP * GD) * EB, EB)
        pltpu.sync_copy(col_hbm.at[pl.ds(off, EB)], idx[0])
        pltpu.sync_copy(m_hbm.at[pl.ds(off, EB)], mb[0])
        pltpu.sync_copy(mb[0], acc_sh.at[idx[0]], add=True)

        plsc.subcore_barrier()
        zoff = 0
        while zoff < ZR:
            step = min(EB, ZR - zoff)
            pltpu.sync_copy(acc_sh.at[pl.ds(s * ZR + zoff, step)],
                            mb[0].at[pl.ds(0, step)])
            pltpu.sync_copy(mb[0].at[pl.ds(0, step)],
                            out_hbm.at[c, pl.ds(s * ZR + zoff, step)])
            zoff += step

    @functools.partial(
        pl.kernel,
        out_type=jax.ShapeDtypeStruct((NC, N_PAD, D), F32),
        mesh=mesh,
        scratch_types=(
            [pltpu.VMEM((EB,), I32) for _ in range(GD)]
            + [pltpu.VMEM((EB, D), F32)]
            + [pltpu.VMEM_SHARED((N_PAD, D), F32)]
            + [pltpu.SemaphoreType.DMA for _ in range(2 * GD)]
        ),
    )
    def count(col_hbm, ones_hbm, zeros_hbm, out_hbm, *scr):
        idx = scr[0:GD]
        ones_v = scr[GD]
        acc_sh = scr[GD + 1]
        lsem = scr[GD + 2:GD + 2 + GD]
        ssem = scr[GD + 2 + GD:GD + 2 + 2 * GD]
        c = lax.axis_index("c")
        s = lax.axis_index("s")
        base = (c * NS + s) * BPW

        pltpu.sync_copy(zeros_hbm, ones_v)
        zoff = 0
        while zoff < ZR:
            step = min(EB, ZR - zoff)
            pltpu.sync_copy(ones_v.at[pl.ds(0, step)],
                            acc_sh.at[pl.ds(s * ZR + zoff, step)])
            zoff += step
        pltpu.sync_copy(ones_hbm, ones_v)
        plsc.subcore_barrier()

        def fire_load(k, blk):
            off = pl.multiple_of((base + blk) * EB, EB)
            pltpu.async_copy(col_hbm.at[pl.ds(off, EB)], idx[k], lsem[k])

        def wait_load(k, blk):
            off = pl.multiple_of((base + blk) * EB, EB)
            pltpu.make_async_copy(col_hbm.at[pl.ds(off, EB)], idx[k], lsem[k]).wait()

        for k in range(GD):
            fire_load(k, k)

        def body(i, carry):
            for k in range(GD):
                wait_load(k, i * GD + k)
                pltpu.async_copy(ones_v, acc_sh.at[idx[k]], ssem[k], add=True)
            for k in range(GD):
                pltpu.make_async_copy(ones_v, acc_sh.at[idx[k]], ssem[k]).wait()

                @pl.when(i < NGRP - 1)
                def _():
                    fire_load(k, (i + 1) * GD + k)
            return carry

        lax.fori_loop(0, NGRP, body, 0)
        off = pl.multiple_of((base + NGRP * GD) * EB, EB)
        pltpu.sync_copy(col_hbm.at[pl.ds(off, EB)], idx[0])
        pltpu.sync_copy(ones_v, acc_sh.at[idx[0]], add=True)

        plsc.subcore_barrier()
        zoff = 0
        while zoff < ZR:
            step = min(EB, ZR - zoff)
            pltpu.sync_copy(acc_sh.at[pl.ds(s * ZR + zoff, step)],
                            ones_v.at[pl.ds(0, step)])
            pltpu.sync_copy(ones_v.at[pl.ds(0, step)],
                            out_hbm.at[c, pl.ds(s * ZR + zoff, step)])
            zoff += step

    return gather2, scatter_add, count


def _sc_gather2(x, idx2):
    return _sc_build()[0](x, idx2)


def _sc_scatter_add(m, col_s, zeros):
    return _sc_build()[1](m, col_s, zeros)


def _sc_count(col_s, ones, zeros):
    return _sc_build()[2](col_s, ones, zeros)


BF16 = jnp.bfloat16
U32 = jnp.uint32
U16 = jnp.uint16


def _unpack2(p):
    u = lax.bitcast_convert_type(p, U32)
    lo = lax.bitcast_convert_type((u & 0xFFFF).astype(U16), BF16)
    hi = lax.bitcast_convert_type((u >> 16).astype(U16), BF16)
    return jnp.concatenate([lo, hi], axis=1)


def _pack2(x16):
    lo = lax.bitcast_convert_type(x16[:, :DP], U16).astype(U32)
    hi = lax.bitcast_convert_type(x16[:, DP:], U16).astype(U32)
    return lax.bitcast_convert_type(lo | (hi << 16), F32)


def _edge_body(xr, xd, ea, wes, wed, wee, be, wn1x, wn1e, bn1, e_ref, m_ref):
    xr16 = _unpack2(xr[...])
    xd16 = _unpack2(xd[...])
    acc = jnp.dot(xr16, wes[...], preferred_element_type=F32)
    acc += jnp.dot(xd16, wed[...], preferred_element_type=F32)
    acc += jnp.dot(ea[...], wee[...], preferred_element_type=F32)
    e16 = jnp.maximum(acc + be[...], 0.0).astype(BF16)
    e_ref[...] = e16
    m = jnp.dot(xr16, wn1x[...], preferred_element_type=F32)
    m += jnp.dot(e16, wn1e[...], preferred_element_type=F32)
    m_ref[...] = jnp.maximum(m + bn1[...], 0.0)


def _tc_edge(xr, xd, ea, wes, wed, wee, be, wn1x, wn1e, bn1):
    ei = ea.shape[1]
    cst = lambda i: (0, 0)
    row = lambda i: (i, 0)
    return pl.pallas_call(
        _edge_body,
        grid=(GE,),
        in_specs=[
            pl.BlockSpec((TE, DP), row),
            pl.BlockSpec((TE, DP), row),
            pl.BlockSpec((TE, ei), row),
            pl.BlockSpec((D, EO), cst),
            pl.BlockSpec((D, EO), cst),
            pl.BlockSpec((ei, EO), cst),
            pl.BlockSpec((1, EO), cst),
            pl.BlockSpec((D, D), cst),
            pl.BlockSpec((EO, D), cst),
            pl.BlockSpec((1, D), cst),
        ],
        out_specs=[pl.BlockSpec((TE, EO), row), pl.BlockSpec((TE, D), row)],
        out_shape=[jax.ShapeDtypeStruct((E_PAD, EO), BF16),
                   jax.ShapeDtypeStruct((E_PAD, D), F32)],
    )(xr, xd, ea, wes, wed, wee, be, wn1x, wn1e, bn1)


def _node_body(x, p0, p1, invc, b3, wn2x, wn2a, bn2, xn_ref, gs_ref, gc_ref):
    i = pl.program_id(0)
    agg = (p0[0] + p1[0]) * invc[...]
    xn = jnp.dot(_unpack2(x[...]), wn2x[...], preferred_element_type=F32)
    xn += jnp.dot(agg, wn2a[...], preferred_element_type=F32)
    xn = jnp.maximum(xn + bn2[...], 0.0)
    xn_ref[...] = _pack2(xn.astype(BF16))
    seg = b3[0]
    ids = lax.broadcasted_iota(I32, (NG, TN), 0)
    oh = (ids == seg).astype(F32)

    @pl.when(i == 0)
    def _init():
        gs_ref[...] = jnp.zeros_like(gs_ref)
        gc_ref[...] = jnp.zeros_like(gc_ref)

    gs_ref[...] += jnp.dot(oh, xn, preferred_element_type=F32)
    gc_ref[...] += jnp.broadcast_to(jnp.sum(oh, axis=1, keepdims=True), (NG, D))


def _tc_node(x, parts, invc, b3, wn2x, wn2a, bn2):
    cst = lambda i: (0, 0)
    return pl.pallas_call(
        _node_body,
        grid=(GN,),
        in_specs=[
            pl.BlockSpec((TN, DP), lambda i: (i, 0)),
            pl.BlockSpec((1, TN, D), lambda i: (0, i, 0)),
            pl.BlockSpec((1, TN, D), lambda i: (1, i, 0)),
            pl.BlockSpec((TN, D), lambda i: (i, 0)),
            pl.BlockSpec((1, 1, TN), lambda i: (i, 0, 0)),
            pl.BlockSpec((D, D), cst),
            pl.BlockSpec((D, D), cst),
            pl.BlockSpec((1, D), cst),
        ],
        out_specs=[pl.BlockSpec((TN, DP), lambda i: (i, 0)),
                   pl.BlockSpec((NG, D), cst),
                   pl.BlockSpec((NG, D), cst)],
        out_shape=[jax.ShapeDtypeStruct((N_PAD, DP), F32),
                   jax.ShapeDtypeStruct((NG, D), F32),
                   jax.ShapeDtypeStruct((NG, D), F32)],
    )(x, parts, parts, invc, b3, wn2x, wn2a, bn2)


def _inv_body(c0, c1, out):
    out[...] = 1.0 / jnp.maximum(c0[0] + c1[0], 1.0)


def _tc_invcnt(cnt_parts):
    return pl.pallas_call(
        _inv_body,
        grid=(GN,),
        in_specs=[pl.BlockSpec((1, TN, D), lambda i: (0, i, 0)),
                  pl.BlockSpec((1, TN, D), lambda i: (1, i, 0))],
        out_specs=pl.BlockSpec((TN, D), lambda i: (i, 0)),
        out_shape=jax.ShapeDtypeStruct((N, D), F32),
    )(cnt_parts, cnt_parts)


def _tc_global(gs, gc, u, wgu, wgm, bg):
    go = wgm.shape[1]

    if u is None:
        def body(gs_r, gc_r, wgm_r, bg_r, out):
            gm = gs_r[...] / jnp.maximum(gc_r[...], 1.0)
            out[...] = jnp.dot(gm, wgm_r[...], preferred_element_type=F32) + bg_r[...]
        args = (gs, gc, wgm, bg)
    else:
        def body(gs_r, gc_r, u_r, wgu_r, wgm_r, bg_r, out):
            gm = gs_r[...] / jnp.maximum(gc_r[...], 1.0)
            acc = jnp.dot(gm, wgm_r[...], preferred_element_type=F32)
            acc += jnp.dot(u_r[...], wgu_r[...], preferred_element_type=F32)
            out[...] = acc + bg_r[...]
        args = (gs, gc, u, wgu, wgm, bg)

    return pl.pallas_call(
        body,
        out_shape=jax.ShapeDtypeStruct((NG, go), F32),
    )(*args)


def _head_body(rg, pg, wr, br, wp, bp, wyr, wyp, by, out):
    a = jnp.dot(rg[...], wr[...], preferred_element_type=F32) + br[...]
    a = jnp.where(a > 0, a, jnp.exp(jnp.minimum(a, 0.0)) - 1.0)
    b = jnp.dot(pg[...], wp[...], preferred_element_type=F32) + bp[...]
    b = jnp.where(b > 0, b, jnp.exp(jnp.minimum(b, 0.0)) - 1.0)
    y = jnp.dot(a, wyr[...], preferred_element_type=F32)
    y += jnp.dot(b, wyp[...], preferred_element_type=F32)
    y = jax.nn.sigmoid(y + by[...]) * 100.0
    out[...] = jnp.broadcast_to(y[:, :1], (NG, D))


def _tc_head(rg, pg, wr, br, wp, bp, wyr, wyp, by):
    return pl.pallas_call(
        _head_body,
        out_shape=jax.ShapeDtypeStruct((NG, D), F32),
    )(rg, pg, wr, br, wp, bp, wyr, wyp, by)


def _meta_layer(x, ea, u, idx2, col_s, invc, b3, zeros, prm):
    (wes, wed, wee, be, wn1x, wn1e, bn1, wn2x, wn2a, bn2, wgu, wgm, bg) = prm
    xr, xd = _sc_gather2(x, idx2)
    e, m = _tc_edge(xr, xd, ea, wes, wed, wee, be, wn1x, wn1e, bn1)
    parts = _sc_scatter_add(m, col_s, zeros)
    xn, gs, gc = _tc_node(x, parts, invc, b3, wn2x, wn2a, bn2)
    un = _tc_global(gs, gc, u, wgu, wgm, bg)
    return xn, e, un


def _split_layer(We, be, Wn1, bn1, Wn2, bn2, Wg, bg, gi, fold_bn=None):
    wes, wed, wee = We[:D], We[D:2 * D], We[2 * D:]
    wn1x, wn1e = Wn1[:D], Wn1[D:]
    wn2x, wn2a = Wn2[:D], Wn2[D:]
    if fold_bn is not None:
        nsc, nsh, esc, esh = fold_bn
        be = be + nsh @ wes + nsh @ wed + esh @ wee
        wes = nsc[:, None] * wes
        wed = nsc[:, None] * wed
        wee = esc[:, None] * wee
        bn1 = bn1 + nsh @ wn1x
        wn1x = nsc[:, None] * wn1x
        bn2 = bn2 + nsh @ wn2x
        wn2x = nsc[:, None] * wn2x
    wgu, wgm = (Wg[:gi], Wg[gi:]) if gi > 0 else (None, Wg)
    b16 = jnp.bfloat16
    return (wes.astype(b16), wed.astype(b16), wee.astype(b16), be[None, :],
            wn1x.astype(b16), wn1e.astype(b16), bn1[None, :],
            wn2x.astype(b16), wn2a, bn2[None, :], wgu, wgm, bg[None, :])


def kernel(rx, re, rg, px, pe, rc, pc, rb, pb, bnn_g, bnn_b, bnn_m, bnn_v, bne_g, bne_b, bne_m, bne_v, We_r1, be_r1, Wn1_r1, bn1_r1, Wn2_r1, bn2_r1, Wg_r1, bg_r1, We_r2, be_r2, Wn1_r2, bn1_r2, Wn2_r2, bn2_r2, Wg_r2, bg_r2, We_r3, be_r3, Wn1_r3, bn1_r3, Wn2_r3, bn2_r3, Wg_r3, bg_r3, We_r4, be_r4, Wn1_r4, bn1_r4, Wn2_r4, bn2_r4, Wg_r4, bg_r4, We_r5, be_r5, Wn1_r5, bn1_r5, Wn2_r5, bn2_r5, Wg_r5, bg_r5, We_p1, be_p1, Wn1_p1, bn1_p1, Wn2_p1, bn2_p1, Wg_p1, bg_p1, We_p2, be_p2, Wn1_p2, bn1_p2, Wn2_p2, bn2_p2, Wg_p2, bg_p2, We_p3, be_p3, Wn1_p3, bn1_p3, Wn2_p3, bn2_p3, Wg_p3, bg_p3, W_rlin, b_rlin, W_plin, b_plin, W_y, b_y):
    pad = E_PAD - E
    zpad = jnp.zeros((pad,), I32)
    npad = jnp.full((pad,), N, I32)
    rrow = jnp.concatenate([rc[0].astype(I32), zpad])
    rcol_g = jnp.concatenate([rc[1].astype(I32), zpad])
    rcol_s = jnp.concatenate([rc[1].astype(I32), npad])
    prow = jnp.concatenate([pc[0].astype(I32), zpad])
    pcol_g = jnp.concatenate([pc[1].astype(I32), zpad])
    pcol_s = jnp.concatenate([pc[1].astype(I32), npad])
    ridx2 = jnp.stack([rrow.reshape(NBLK, EB), rcol_g.reshape(NBLK, EB)], axis=1)
    pidx2 = jnp.stack([prow.reshape(NBLK, EB), pcol_g.reshape(NBLK, EB)], axis=1)
    rb3 = rb.astype(I32).reshape(GN, 1, TN)
    pb3 = pb.astype(I32).reshape(GN, 1, TN)

    nsc = bnn_g / jnp.sqrt(bnn_v + EPS)
    nsh = bnn_b - bnn_m * nsc
    esc = bne_g / jnp.sqrt(bne_v + EPS)
    esh = bne_b - bne_m * esc
    fold = (nsc, nsh, esc, esh)

    re_p = jnp.concatenate([re, jnp.zeros((pad, re.shape[1]), F32)]).astype(jnp.bfloat16)
    pe_p = jnp.concatenate([pe, jnp.zeros((pad, pe.shape[1]), F32)]).astype(jnp.bfloat16)

    zeros = jnp.zeros((EB, D), F32)
    ones = jnp.ones((EB, D), F32)

    def pack_host(x):
        x16 = x.astype(jnp.bfloat16)
        lo = lax.bitcast_convert_type(x16[:, :DP], jnp.uint16).astype(jnp.uint32)
        hi = lax.bitcast_convert_type(x16[:, DP:], jnp.uint16).astype(jnp.uint32)
        return lax.bitcast_convert_type(lo | (hi << 16), F32)

    rx_p = jnp.pad(pack_host(rx), ((0, N_PAD - N), (0, 0)))
    px_p = jnp.pad(pack_host(px), ((0, N_PAD - N), (0, 0)))

    r_layers = [
        _split_layer(We_r1, be_r1, Wn1_r1, bn1_r1, Wn2_r1, bn2_r1, Wg_r1, bg_r1, 32, fold),
        _split_layer(We_r2, be_r2, Wn1_r2, bn1_r2, Wn2_r2, bn2_r2, Wg_r2, bg_r2, 128),
        _split_layer(We_r3, be_r3, Wn1_r3, bn1_r3, Wn2_r3, bn2_r3, Wg_r3, bg_r3, 128),
        _split_layer(We_r4, be_r4, Wn1_r4, bn1_r4, Wn2_r4, bn2_r4, Wg_r4, bg_r4, 128),
        _split_layer(We_r5, be_r5, Wn1_r5, bn1_r5, Wn2_r5, bn2_r5, Wg_r5, bg_r5, 128),
    ]
    p_layers = [
        _split_layer(We_p1, be_p1, Wn1_p1, bn1_p1, Wn2_p1, bn2_p1, Wg_p1, bg_p1, 0, fold),
        _split_layer(We_p2, be_p2, Wn1_p2, bn1_p2, Wn2_p2, bn2_p2, Wg_p2, bg_p2, 128),
        _split_layer(We_p3, be_p3, Wn1_p3, bn1_p3, Wn2_p3, bn2_p3, Wg_p3, bg_p3, 128),
    ]

    r_invc = _tc_invcnt(_sc_count(rcol_s, ones, zeros))
    p_invc = _tc_invcnt(_sc_count(pcol_s, ones, zeros))

    rst = (rx_p, re_p, rg)
    pst = (px_p, pe_p, None)
    for i in range(5):
        rst = _meta_layer(*rst, ridx2, rcol_s, r_invc, rb3, zeros, r_layers[i])
        if i < 3:
            pst = _meta_layer(*pst, pidx2, pcol_s, p_invc, pb3, zeros, p_layers[i])
    rg_out = rst[2]
    pg_out = pst[2]

    y = _tc_head(rg_out, pg_out,
                 W_rlin, b_rlin[None, :], W_plin, b_plin[None, :],
                 W_y[:256], W_y[256:], b_y[None, :])
    return y[:, :1]


# speedup vs baseline: 1.6509x; 1.0833x over previous
import functools

import jax
import jax.numpy as jnp
from jax import lax
from jax.experimental import pallas as pl
from jax.experimental.pallas import tpu as pltpu
from jax.experimental.pallas import tpu_sc as plsc

F32 = jnp.float32
I32 = jnp.int32

N = 10000
E = 160000
D = 128
EO = 512
NG = 16
EPS = 1e-5

NC = 2
NS = 16
NW = NC * NS
EB = 128
E_PAD = 163840
NBLK = E_PAD // EB
BPW = E_PAD // (NW * EB)
N_PAD = 10112
ZR = N_PAD // NS

DP = 64

TE = 2560
GE = E_PAD // TE
TN = 1000
GN = N // TN


@functools.cache
def _sc_build():
    mesh = plsc.VectorSubcoreMesh(core_axis_name="c", subcore_axis_name="s")
    GD = 3
    NGRP = BPW // GD
    GDG = 5
    NGRPG = BPW // GDG

    @functools.partial(
        pl.kernel,
        out_type=(jax.ShapeDtypeStruct((E_PAD, DP), F32),
                  jax.ShapeDtypeStruct((E_PAD, DP), F32)),
        mesh=mesh,
        compiler_params=pltpu.CompilerParams(use_tc_tiling_on_sc=False),
        scratch_types=(
            [pltpu.VMEM((2, EB), I32) for _ in range(GDG)]
            + [pltpu.VMEM((2 * EB, DP), F32) for _ in range(GDG)]
            + [pltpu.VMEM_SHARED((N_PAD, DP), F32)]
            + [pltpu.SemaphoreType.DMA for _ in range(2 * GDG)]
        ),
    )
    def gather2(x_hbm, idx2_hbm, outr_hbm, outc_hbm, *scr):
        ib = scr[0:GDG]
        bd = scr[GDG:2 * GDG]
        xs_sh = scr[2 * GDG]
        sems = scr[2 * GDG + 1:4 * GDG + 1]
        wid = lax.axis_index("c") * NS + lax.axis_index("s")
        base = wid * BPW
        s = lax.axis_index("s")
        pltpu.sync_copy(x_hbm.at[pl.ds(s * ZR, ZR)], xs_sh.at[pl.ds(s * ZR, ZR)])
        plsc.subcore_barrier()

        def do_group(first_blk, nk):
            for k in range(nk):
                g = base + first_blk + k
                pltpu.sync_copy(idx2_hbm.at[g], ib[k])
                pltpu.async_copy(xs_sh.at[ib[k].at[0]], bd[k].at[pl.ds(0, EB)],
                                 sems[2 * k])
                pltpu.async_copy(xs_sh.at[ib[k].at[1]], bd[k].at[pl.ds(EB, EB)],
                                 sems[2 * k + 1])
            for k in range(nk):
                g = base + first_blk + k
                off = pl.multiple_of(g * EB, EB)
                pltpu.make_async_copy(xs_sh.at[ib[k].at[0]],
                                      bd[k].at[pl.ds(0, EB)], sems[2 * k]).wait()
                pltpu.make_async_copy(xs_sh.at[ib[k].at[1]],
                                      bd[k].at[pl.ds(EB, EB)], sems[2 * k + 1]).wait()
                pltpu.sync_copy(bd[k].at[pl.ds(0, EB)], outr_hbm.at[pl.ds(off, EB)])
                pltpu.sync_copy(bd[k].at[pl.ds(EB, EB)], outc_hbm.at[pl.ds(off, EB)])

        def body(i, carry):
            do_group(i * GDG, GDG)
            return carry

        lax.fori_loop(0, NGRPG, body, 0)
        if BPW % GDG:
            do_group(NGRPG * GDG, BPW % GDG)

    @functools.partial(
        pl.kernel,
        out_type=jax.ShapeDtypeStruct((NC, N_PAD, D), F32),
        mesh=mesh,
        scratch_types=(
            [pltpu.VMEM((EB,), I32) for _ in range(GD)]
            + [pltpu.VMEM((EB, D), F32) for _ in range(GD)]
            + [pltpu.VMEM_SHARED((N_PAD, D), F32)]
            + [pltpu.SemaphoreType.DMA for _ in range(2 * GD)]
        ),
    )
    def scatter_add(m_hbm, col_hbm, zeros_hbm, out_hbm, *scr):
        idx = scr[0:GD]
        mb = scr[GD:2 * GD]
        acc_sh = scr[2 * GD]
        lsem = scr[2 * GD + 1:2 * GD + 1 + GD]
        ssem = scr[2 * GD + 1 + GD:2 * GD + 1 + 2 * GD]
        c = lax.axis_index("c")
        s = lax.axis_index("s")
        base = (c * NS + s) * BPW

        pltpu.sync_copy(zeros_hbm, mb[0])
        zoff = 0
        while zoff < ZR:
            step = min(EB, ZR - zoff)
            pltpu.sync_copy(mb[0].at[pl.ds(0, step)],
                            acc_sh.at[pl.ds(s * ZR + zoff, step)])
            zoff += step
        plsc.subcore_barrier()

        def fire_loads(k, blk):
            off = pl.multiple_of((base + blk) * EB, EB)
            pltpu.async_copy(col_hbm.at[pl.ds(off, EB)], idx[k], lsem[k])
            pltpu.async_copy(m_hbm.at[pl.ds(off, EB)], mb[k], lsem[k])

        def wait_loads(k, blk):
            off = pl.multiple_of((base + blk) * EB, EB)
            pltpu.make_async_copy(col_hbm.at[pl.ds(off, EB)], idx[k], lsem[k]).wait()
            pltpu.make_async_copy(m_hbm.at[pl.ds(off, EB)], mb[k], lsem[k]).wait()

        for k in range(GD):
            fire_loads(k, k)

        def body(i, carry):
            for k in range(GD):
                wait_loads(k, i * GD + k)
                pltpu.async_copy(mb[k], acc_sh.at[idx[k]], ssem[k], add=True)
            for k in range(GD):
                pltpu.make_async_copy(mb[k], acc_sh.at[idx[k]], ssem[k]).wait()

                @pl.when(i < NGRP - 1)
                def _():
                    fire_loads(k, (i + 1) * GD + k)
            return carry

        lax.fori_loop(0, NGRP, body, 0)
        off = pl.multiple_of((base + NGRP * GD) * EB, EB)
        pltpu.sync_copy(col_hbm.at[pl.ds(off, EB)], idx[0])
        pltpu.sync_copy(m_hbm.at[pl.ds(off, EB)], mb[0])
        pltpu.sync_copy(mb[0], acc_sh.at[idx[0]], add=True)

        plsc.subcore_barrier()
        zoff = 0
        while zoff < ZR:
            step = min(EB, ZR - zoff)
            pltpu.sync_copy(acc_sh.at[pl.ds(s * ZR + zoff, step)],
                            mb[0].at[pl.ds(0, step)])
            pltpu.sync_copy(mb[0].at[pl.ds(0, step)],
                            out_hbm.at[c, pl.ds(s * ZR + zoff, step)])
            zoff += step

    @functools.partial(
        pl.kernel,
        out_type=jax.ShapeDtypeStruct((NC, N_PAD, D), F32),
        mesh=mesh,
        scratch_types=(
            [pltpu.VMEM((EB,), I32) for _ in range(GD)]
            + [pltpu.VMEM((EB, D), F32)]
            + [pltpu.VMEM_SHARED((N_PAD, D), F32)]
            + [pltpu.SemaphoreType.DMA for _ in range(2 * GD)]
        ),
    )
    def count(col_hbm, ones_hbm, zeros_hbm, out_hbm, *scr):
        idx = scr[0:GD]
        ones_v = scr[GD]
        acc_sh = scr[GD + 1]
        lsem = scr[GD + 2:GD + 2 + GD]
        ssem = scr[GD + 2 + GD:GD + 2 + 2 * GD]
        c = lax.axis_index("c")
        s = lax.axis_index("s")
        base = (c * NS + s) * BPW

        pltpu.sync_copy(zeros_hbm, ones_v)
        zoff = 0
        while zoff < ZR:
            step = min(EB, ZR - zoff)
            pltpu.sync_copy(ones_v.at[pl.ds(0, step)],
                            acc_sh.at[pl.ds(s * ZR + zoff, step)])
            zoff += step
        pltpu.sync_copy(ones_hbm, ones_v)
        plsc.subcore_barrier()

        def fire_load(k, blk):
            off = pl.multiple_of((base + blk) * EB, EB)
            pltpu.async_copy(col_hbm.at[pl.ds(off, EB)], idx[k], lsem[k])

        def wait_load(k, blk):
            off = pl.multiple_of((base + blk) * EB, EB)
            pltpu.make_async_copy(col_hbm.at[pl.ds(off, EB)], idx[k], lsem[k]).wait()

        for k in range(GD):
            fire_load(k, k)

        def body(i, carry):
            for k in range(GD):
                wait_load(k, i * GD + k)
                pltpu.async_copy(ones_v, acc_sh.at[idx[k]], ssem[k], add=True)
            for k in range(GD):
                pltpu.make_async_copy(ones_v, acc_sh.at[idx[k]], ssem[k]).wait()

                @pl.when(i < NGRP - 1)
                def _():
                    fire_load(k, (i + 1) * GD + k)
            return carry

        lax.fori_loop(0, NGRP, body, 0)
        off = pl.multiple_of((base + NGRP * GD) * EB, EB)
        pltpu.sync_copy(col_hbm.at[pl.ds(off, EB)], idx[0])
        pltpu.sync_copy(ones_v, acc_sh.at[idx[0]], add=True)

        plsc.subcore_barrier()
        zoff = 0
        while zoff < ZR:
            step = min(EB, ZR - zoff)
            pltpu.sync_copy(acc_sh.at[pl.ds(s * ZR + zoff, step)],
                            ones_v.at[pl.ds(0, step)])
            pltpu.sync_copy(ones_v.at[pl.ds(0, step)],
                            out_hbm.at[c, pl.ds(s * ZR + zoff, step)])
            zoff += step

    return gather2, scatter_add, count


def _sc_gather2(x, idx2):
    return _sc_build()[0](x, idx2)


def _sc_scatter_add(m, col_s, zeros):
    return _sc_build()[1](m, col_s, zeros)


def _sc_count(col_s, ones, zeros):
    return _sc_build()[2](col_s, ones, zeros)


BF16 = jnp.bfloat16
U32 = jnp.uint32
U16 = jnp.uint16


def _unpack2(p):
    u = lax.bitcast_convert_type(p, U32)
    lo = lax.bitcast_convert_type((u & 0xFFFF).astype(U16), BF16)
    hi = lax.bitcast_convert_type((u >> 16).astype(U16), BF16)
    return jnp.concatenate([lo, hi], axis=1)


def _pack2(x16):
    lo = lax.bitcast_convert_type(x16[:, :DP], U16).astype(U32)
    hi = lax.bitcast_convert_type(x16[:, DP:], U16).astype(U32)
    return lax.bitcast_convert_type(lo | (hi << 16), F32)


def _edge_body(xr, xd, ea, wes, wed, wee, be, wn1x, wn1e, bn1, e_ref, m_ref):
    xr16 = _unpack2(xr[...])
    xd16 = _unpack2(xd[...])
    acc = jnp.dot(xr16, wes[...], preferred_element_type=F32)
    acc += jnp.dot(xd16, wed[...], preferred_element_type=F32)
    acc += jnp.dot(ea[...], wee[...], preferred_element_type=F32)
    e16 = jnp.maximum(acc + be[...], 0.0).astype(BF16)
    e_ref[...] = e16
    m = jnp.dot(xr16, wn1x[...], preferred_element_type=F32)
    m += jnp.dot(e16, wn1e[...], preferred_element_type=F32)
    m_ref[...] = jnp.maximum(m + bn1[...], 0.0)


def _tc_edge(xr, xd, ea, wes, wed, wee, be, wn1x, wn1e, bn1):
    ei = ea.shape[1]
    cst = lambda i: (0, 0)
    row = lambda i: (i, 0)
    return pl.pallas_call(
        _edge_body,
        grid=(GE,),
        in_specs=[
            pl.BlockSpec((TE, DP), row),
            pl.BlockSpec((TE, DP), row),
            pl.BlockSpec((TE, ei), row),
            pl.BlockSpec((D, EO), cst),
            pl.BlockSpec((D, EO), cst),
            pl.BlockSpec((ei, EO), cst),
            pl.BlockSpec((1, EO), cst),
            pl.BlockSpec((D, D), cst),
            pl.BlockSpec((EO, D), cst),
            pl.BlockSpec((1, D), cst),
        ],
        out_specs=[pl.BlockSpec((TE, EO), row), pl.BlockSpec((TE, D), row)],
        out_shape=[jax.ShapeDtypeStruct((E_PAD, EO), BF16),
                   jax.ShapeDtypeStruct((E_PAD, D), F32)],
    )(xr, xd, ea, wes, wed, wee, be, wn1x, wn1e, bn1)


def _node_body(x, p0, p1, invc, b3, wn2x, wn2a, bn2, xn_ref, gs_ref, gc_ref):
    i = pl.program_id(0)
    agg = (p0[0] + p1[0]) * invc[...]
    xn = jnp.dot(_unpack2(x[...]), wn2x[...], preferred_element_type=F32)
    xn += jnp.dot(agg, wn2a[...], preferred_element_type=F32)
    xn = jnp.maximum(xn + bn2[...], 0.0)
    xn_ref[...] = _pack2(xn.astype(BF16))
    seg = b3[0]
    ids = lax.broadcasted_iota(I32, (NG, TN), 0)
    oh = (ids == seg).astype(F32)

    @pl.when(i == 0)
    def _init():
        gs_ref[...] = jnp.zeros_like(gs_ref)
        gc_ref[...] = jnp.zeros_like(gc_ref)

    gs_ref[...] += jnp.dot(oh, xn, preferred_element_type=F32)
    gc_ref[...] += jnp.broadcast_to(jnp.sum(oh, axis=1, keepdims=True), (NG, D))


def _tc_node(x, parts, invc, b3, wn2x, wn2a, bn2):
    cst = lambda i: (0, 0)
    return pl.pallas_call(
        _node_body,
        grid=(GN,),
        in_specs=[
            pl.BlockSpec((TN, DP), lambda i: (i, 0)),
            pl.BlockSpec((1, TN, D), lambda i: (0, i, 0)),
            pl.BlockSpec((1, TN, D), lambda i: (1, i, 0)),
            pl.BlockSpec((TN, D), lambda i: (i, 0)),
            pl.BlockSpec((1, 1, TN), lambda i: (i, 0, 0)),
            pl.BlockSpec((D, D), cst),
            pl.BlockSpec((D, D), cst),
            pl.BlockSpec((1, D), cst),
        ],
        out_specs=[pl.BlockSpec((TN, DP), lambda i: (i, 0)),
                   pl.BlockSpec((NG, D), cst),
                   pl.BlockSpec((NG, D), cst)],
        out_shape=[jax.ShapeDtypeStruct((N_PAD, DP), F32),
                   jax.ShapeDtypeStruct((NG, D), F32),
                   jax.ShapeDtypeStruct((NG, D), F32)],
    )(x, parts, parts, invc, b3, wn2x, wn2a, bn2)


def _inv_body(c0, c1, out):
    out[...] = 1.0 / jnp.maximum(c0[0] + c1[0], 1.0)


def _tc_invcnt(cnt_parts):
    return pl.pallas_call(
        _inv_body,
        grid=(GN,),
        in_specs=[pl.BlockSpec((1, TN, D), lambda i: (0, i, 0)),
                  pl.BlockSpec((1, TN, D), lambda i: (1, i, 0))],
        out_specs=pl.BlockSpec((TN, D), lambda i: (i, 0)),
        out_shape=jax.ShapeDtypeStruct((N, D), F32),
    )(cnt_parts, cnt_parts)


def _tc_global(gs, gc, u, wgu, wgm, bg):
    go = wgm.shape[1]

    if u is None:
        def body(gs_r, gc_r, wgm_r, bg_r, out):
            gm = gs_r[...] / jnp.maximum(gc_r[...], 1.0)
            out[...] = jnp.dot(gm, wgm_r[...], preferred_element_type=F32) + bg_r[...]
        args = (gs, gc, wgm, bg)
    else:
        def body(gs_r, gc_r, u_r, wgu_r, wgm_r, bg_r, out):
            gm = gs_r[...] / jnp.maximum(gc_r[...], 1.0)
            acc = jnp.dot(gm, wgm_r[...], preferred_element_type=F32)
            acc += jnp.dot(u_r[...], wgu_r[...], preferred_element_type=F32)
            out[...] = acc + bg_r[...]
        args = (gs, gc, u, wgu, wgm, bg)

    return pl.pallas_call(
        body,
        out_shape=jax.ShapeDtypeStruct((NG, go), F32),
    )(*args)


def _head_body(rg, pg, wr, br, wp, bp, wyr, wyp, by, out):
    a = jnp.dot(rg[...], wr[...], preferred_element_type=F32) + br[...]
    a = jnp.where(a > 0, a, jnp.exp(jnp.minimum(a, 0.0)) - 1.0)
    b = jnp.dot(pg[...], wp[...], preferred_element_type=F32) + bp[...]
    b = jnp.where(b > 0, b, jnp.exp(jnp.minimum(b, 0.0)) - 1.0)
    y = jnp.dot(a, wyr[...], preferred_element_type=F32)
    y += jnp.dot(b, wyp[...], preferred_element_type=F32)
    y = jax.nn.sigmoid(y + by[...]) * 100.0
    out[...] = jnp.broadcast_to(y[:, :1], (NG, D))


def _tc_head(rg, pg, wr, br, wp, bp, wyr, wyp, by):
    return pl.pallas_call(
        _head_body,
        out_shape=jax.ShapeDtypeStruct((NG, D), F32),
    )(rg, pg, wr, br, wp, bp, wyr, wyp, by)


def _meta_layer(x, ea, u, idx2, col_s, invc, b3, zeros, prm):
    (wes, wed, wee, be, wn1x, wn1e, bn1, wn2x, wn2a, bn2, wgu, wgm, bg) = prm
    xr, xd = _sc_gather2(x, idx2)
    e, m = _tc_edge(xr, xd, ea, wes, wed, wee, be, wn1x, wn1e, bn1)
    parts = _sc_scatter_add(m, col_s, zeros)
    xn, gs, gc = _tc_node(x, parts, invc, b3, wn2x, wn2a, bn2)
    un = _tc_global(gs, gc, u, wgu, wgm, bg)
    return xn, e, un


def _split_layer(We, be, Wn1, bn1, Wn2, bn2, Wg, bg, gi, fold_bn=None):
    wes, wed, wee = We[:D], We[D:2 * D], We[2 * D:]
    wn1x, wn1e = Wn1[:D], Wn1[D:]
    wn2x, wn2a = Wn2[:D], Wn2[D:]
    if fold_bn is not None:
        nsc, nsh, esc, esh = fold_bn
        be = be + nsh @ wes + nsh @ wed + esh @ wee
        wes = nsc[:, None] * wes
        wed = nsc[:, None] * wed
        wee = esc[:, None] * wee
        bn1 = bn1 + nsh @ wn1x
        wn1x = nsc[:, None] * wn1x
        bn2 = bn2 + nsh @ wn2x
        wn2x = nsc[:, None] * wn2x
    wgu, wgm = (Wg[:gi], Wg[gi:]) if gi > 0 else (None, Wg)
    b16 = jnp.bfloat16
    return (wes.astype(b16), wed.astype(b16), wee.astype(b16), be[None, :],
            wn1x.astype(b16), wn1e.astype(b16), bn1[None, :],
            wn2x.astype(b16), wn2a, bn2[None, :], wgu, wgm, bg[None, :])


def kernel(rx, re, rg, px, pe, rc, pc, rb, pb, bnn_g, bnn_b, bnn_m, bnn_v, bne_g, bne_b, bne_m, bne_v, We_r1, be_r1, Wn1_r1, bn1_r1, Wn2_r1, bn2_r1, Wg_r1, bg_r1, We_r2, be_r2, Wn1_r2, bn1_r2, Wn2_r2, bn2_r2, Wg_r2, bg_r2, We_r3, be_r3, Wn1_r3, bn1_r3, Wn2_r3, bn2_r3, Wg_r3, bg_r3, We_r4, be_r4, Wn1_r4, bn1_r4, Wn2_r4, bn2_r4, Wg_r4, bg_r4, We_r5, be_r5, Wn1_r5, bn1_r5, Wn2_r5, bn2_r5, Wg_r5, bg_r5, We_p1, be_p1, Wn1_p1, bn1_p1, Wn2_p1, bn2_p1, Wg_p1, bg_p1, We_p2, be_p2, Wn1_p2, bn1_p2, Wn2_p2, bn2_p2, Wg_p2, bg_p2, We_p3, be_p3, Wn1_p3, bn1_p3, Wn2_p3, bn2_p3, Wg_p3, bg_p3, W_rlin, b_rlin, W_plin, b_plin, W_y, b_y):
    pad = E_PAD - E
    zpad = jnp.zeros((pad,), I32)
    npad = jnp.full((pad,), N, I32)
    rrow = jnp.concatenate([rc[0].astype(I32), zpad])
    rcol_g = jnp.concatenate([rc[1].astype(I32), zpad])
    rcol_s = jnp.concatenate([rc[1].astype(I32), npad])
    prow = jnp.concatenate([pc[0].astype(I32), zpad])
    pcol_g = jnp.concatenate([pc[1].astype(I32), zpad])
    pcol_s = jnp.concatenate([pc[1].astype(I32), npad])
    ridx2 = jnp.stack([rrow.reshape(NBLK, EB), rcol_g.reshape(NBLK, EB)], axis=1)
    pidx2 = jnp.stack([prow.reshape(NBLK, EB), pcol_g.reshape(NBLK, EB)], axis=1)
    rb3 = rb.astype(I32).reshape(GN, 1, TN)
    pb3 = pb.astype(I32).reshape(GN, 1, TN)

    nsc = bnn_g / jnp.sqrt(bnn_v + EPS)
    nsh = bnn_b - bnn_m * nsc
    esc = bne_g / jnp.sqrt(bne_v + EPS)
    esh = bne_b - bne_m * esc
    fold = (nsc, nsh, esc, esh)

    re_p = jnp.concatenate([re, jnp.zeros((pad, re.shape[1]), F32)]).astype(jnp.bfloat16)
    pe_p = jnp.concatenate([pe, jnp.zeros((pad, pe.shape[1]), F32)]).astype(jnp.bfloat16)

    zeros = jnp.zeros((EB, D), F32)
    ones = jnp.ones((EB, D), F32)

    def pack_host(x):
        x16 = x.astype(jnp.bfloat16)
        lo = lax.bitcast_convert_type(x16[:, :DP], jnp.uint16).astype(jnp.uint32)
        hi = lax.bitcast_convert_type(x16[:, DP:], jnp.uint16).astype(jnp.uint32)
        return lax.bitcast_convert_type(lo | (hi << 16), F32)

    rx_p = jnp.pad(pack_host(rx), ((0, N_PAD - N), (0, 0)))
    px_p = jnp.pad(pack_host(px), ((0, N_PAD - N), (0, 0)))

    r_layers = [
        _split_layer(We_r1, be_r1, Wn1_r1, bn1_r1, Wn2_r1, bn2_r1, Wg_r1, bg_r1, 32, fold),
        _split_layer(We_r2, be_r2, Wn1_r2, bn1_r2, Wn2_r2, bn2_r2, Wg_r2, bg_r2, 128),
        _split_layer(We_r3, be_r3, Wn1_r3, bn1_r3, Wn2_r3, bn2_r3, Wg_r3, bg_r3, 128),
        _split_layer(We_r4, be_r4, Wn1_r4, bn1_r4, Wn2_r4, bn2_r4, Wg_r4, bg_r4, 128),
        _split_layer(We_r5, be_r5, Wn1_r5, bn1_r5, Wn2_r5, bn2_r5, Wg_r5, bg_r5, 128),
    ]
    p_layers = [
        _split_layer(We_p1, be_p1, Wn1_p1, bn1_p1, Wn2_p1, bn2_p1, Wg_p1, bg_p1, 0, fold),
        _split_layer(We_p2, be_p2, Wn1_p2, bn1_p2, Wn2_p2, bn2_p2, Wg_p2, bg_p2, 128),
        _split_layer(We_p3, be_p3, Wn1_p3, bn1_p3, Wn2_p3, bn2_p3, Wg_p3, bg_p3, 128),
    ]

    r_invc = _tc_invcnt(_sc_count(rcol_s, ones, zeros))
    p_invc = _tc_invcnt(_sc_count(pcol_s, ones, zeros))

    rst = (rx_p, re_p, rg)
    pst = (px_p, pe_p, None)
    for i in range(5):
        rst = _meta_layer(*rst, ridx2, rcol_s, r_invc, rb3, zeros, r_layers[i])
        if i < 3:
            pst = _meta_layer(*pst, pidx2, pcol_s, p_invc, pb3, zeros, p_layers[i])
    rg_out = rst[2]
    pg_out = pst[2]

    y = _tc_head(rg_out, pg_out,
                 W_rlin, b_rlin[None, :], W_plin, b_plin[None, :],
                 W_y[:256], W_y[256:], b_y[None, :])
    return y[:, :1]


# speedup vs baseline: 1.6724x; 1.0130x over previous
import functools

import jax
import jax.numpy as jnp
from jax import lax
from jax.experimental import pallas as pl
from jax.experimental.pallas import tpu as pltpu
from jax.experimental.pallas import tpu_sc as plsc

F32 = jnp.float32
I32 = jnp.int32

N = 10000
E = 160000
D = 128
EO = 512
NG = 16
EPS = 1e-5

NC = 2
NS = 16
NW = NC * NS
EB = 128
E_PAD = 163840
NBLK = E_PAD // EB
BPW = E_PAD // (NW * EB)
N_PAD = 10112
ZR = N_PAD // NS

DP = 64

TE = 5120
GE = E_PAD // TE
TN = 1000
GN = N // TN


@functools.cache
def _sc_build():
    mesh = plsc.VectorSubcoreMesh(core_axis_name="c", subcore_axis_name="s")
    GD = 3
    NGRP = BPW // GD
    GDG = 5
    NGRPG = BPW // GDG

    @functools.partial(
        pl.kernel,
        out_type=(jax.ShapeDtypeStruct((E_PAD, DP), F32),
                  jax.ShapeDtypeStruct((E_PAD, DP), F32)),
        mesh=mesh,
        compiler_params=pltpu.CompilerParams(use_tc_tiling_on_sc=False),
        scratch_types=(
            [pltpu.VMEM((2, EB), I32) for _ in range(GDG)]
            + [pltpu.VMEM((2 * EB, DP), F32) for _ in range(GDG)]
            + [pltpu.VMEM_SHARED((N_PAD, DP), F32)]
            + [pltpu.SemaphoreType.DMA for _ in range(2 * GDG)]
        ),
    )
    def gather2(x_hbm, idx2_hbm, outr_hbm, outc_hbm, *scr):
        ib = scr[0:GDG]
        bd = scr[GDG:2 * GDG]
        xs_sh = scr[2 * GDG]
        sems = scr[2 * GDG + 1:4 * GDG + 1]
        wid = lax.axis_index("c") * NS + lax.axis_index("s")
        base = wid * BPW
        s = lax.axis_index("s")
        pltpu.sync_copy(x_hbm.at[pl.ds(s * ZR, ZR)], xs_sh.at[pl.ds(s * ZR, ZR)])
        plsc.subcore_barrier()

        def do_group(first_blk, nk):
            for k in range(nk):
                g = base + first_blk + k
                pltpu.sync_copy(idx2_hbm.at[g], ib[k])
                pltpu.async_copy(xs_sh.at[ib[k].at[0]], bd[k].at[pl.ds(0, EB)],
                                 sems[2 * k])
                pltpu.async_copy(xs_sh.at[ib[k].at[1]], bd[k].at[pl.ds(EB, EB)],
                                 sems[2 * k + 1])
            for k in range(nk):
                g = base + first_blk + k
                off = pl.multiple_of(g * EB, EB)
                pltpu.make_async_copy(xs_sh.at[ib[k].at[0]],
                                      bd[k].at[pl.ds(0, EB)], sems[2 * k]).wait()
                pltpu.make_async_copy(xs_sh.at[ib[k].at[1]],
                                      bd[k].at[pl.ds(EB, EB)], sems[2 * k + 1]).wait()
                pltpu.sync_copy(bd[k].at[pl.ds(0, EB)], outr_hbm.at[pl.ds(off, EB)])
                pltpu.sync_copy(bd[k].at[pl.ds(EB, EB)], outc_hbm.at[pl.ds(off, EB)])

        def body(i, carry):
            do_group(i * GDG, GDG)
            return carry

        lax.fori_loop(0, NGRPG, body, 0)
        if BPW % GDG:
            do_group(NGRPG * GDG, BPW % GDG)

    @functools.partial(
        pl.kernel,
        out_type=jax.ShapeDtypeStruct((NC, N_PAD, D), F32),
        mesh=mesh,
        scratch_types=(
            [pltpu.VMEM((EB,), I32) for _ in range(GD)]
            + [pltpu.VMEM((EB, D), F32) for _ in range(GD)]
            + [pltpu.VMEM_SHARED((N_PAD, D), F32)]
            + [pltpu.SemaphoreType.DMA for _ in range(2 * GD)]
        ),
    )
    def scatter_add(m_hbm, col_hbm, zeros_hbm, out_hbm, *scr):
        idx = scr[0:GD]
        mb = scr[GD:2 * GD]
        acc_sh = scr[2 * GD]
        lsem = scr[2 * GD + 1:2 * GD + 1 + GD]
        ssem = scr[2 * GD + 1 + GD:2 * GD + 1 + 2 * GD]
        c = lax.axis_index("c")
        s = lax.axis_index("s")
        base = (c * NS + s) * BPW

        pltpu.sync_copy(zeros_hbm, mb[0])
        zoff = 0
        while zoff < ZR:
            step = min(EB, ZR - zoff)
            pltpu.sync_copy(mb[0].at[pl.ds(0, step)],
                            acc_sh.at[pl.ds(s * ZR + zoff, step)])
            zoff += step
        plsc.subcore_barrier()

        def fire_loads(k, blk):
            off = pl.multiple_of((base + blk) * EB, EB)
            pltpu.async_copy(col_hbm.at[pl.ds(off, EB)], idx[k], lsem[k])
            pltpu.async_copy(m_hbm.at[pl.ds(off, EB)], mb[k], lsem[k])

        def wait_loads(k, blk):
            off = pl.multiple_of((base + blk) * EB, EB)
            pltpu.make_async_copy(col_hbm.at[pl.ds(off, EB)], idx[k], lsem[k]).wait()
            pltpu.make_async_copy(m_hbm.at[pl.ds(off, EB)], mb[k], lsem[k]).wait()

        for k in range(GD):
            fire_loads(k, k)

        def body(i, carry):
            for k in range(GD):
                wait_loads(k, i * GD + k)
                pltpu.async_copy(mb[k], acc_sh.at[idx[k]], ssem[k], add=True)
            for k in range(GD):
                pltpu.make_async_copy(mb[k], acc_sh.at[idx[k]], ssem[k]).wait()

                @pl.when(i < NGRP - 1)
                def _():
                    fire_loads(k, (i + 1) * GD + k)
            return carry

        lax.fori_loop(0, NGRP, body, 0)
        off = pl.multiple_of((base + NGRP * GD) * EB, EB)
        pltpu.sync_copy(col_hbm.at[pl.ds(off, EB)], idx[0])
        pltpu.sync_copy(m_hbm.at[pl.ds(off, EB)], mb[0])
        pltpu.sync_copy(mb[0], acc_sh.at[idx[0]], add=True)

        plsc.subcore_barrier()
        zoff = 0
        while zoff < ZR:
            step = min(EB, ZR - zoff)
            pltpu.sync_copy(acc_sh.at[pl.ds(s * ZR + zoff, step)],
                            mb[0].at[pl.ds(0, step)])
            pltpu.sync_copy(mb[0].at[pl.ds(0, step)],
                            out_hbm.at[c, pl.ds(s * ZR + zoff, step)])
            zoff += step

    @functools.partial(
        pl.kernel,
        out_type=jax.ShapeDtypeStruct((NC, N_PAD, D), F32),
        mesh=mesh,
        scratch_types=(
            [pltpu.VMEM((EB,), I32) for _ in range(GD)]
            + [pltpu.VMEM((EB, D), F32)]
            + [pltpu.VMEM_SHARED((N_PAD, D), F32)]
            + [pltpu.SemaphoreType.DMA for _ in range(2 * GD)]
        ),
    )
    def count(col_hbm, ones_hbm, zeros_hbm, out_hbm, *scr):
        idx = scr[0:GD]
        ones_v = scr[GD]
        acc_sh = scr[GD + 1]
        lsem = scr[GD + 2:GD + 2 + GD]
        ssem = scr[GD + 2 + GD:GD + 2 + 2 * GD]
        c = lax.axis_index("c")
        s = lax.axis_index("s")
        base = (c * NS + s) * BPW

        pltpu.sync_copy(zeros_hbm, ones_v)
        zoff = 0
        while zoff < ZR:
            step = min(EB, ZR - zoff)
            pltpu.sync_copy(ones_v.at[pl.ds(0, step)],
                            acc_sh.at[pl.ds(s * ZR + zoff, step)])
            zoff += step
        pltpu.sync_copy(ones_hbm, ones_v)
        plsc.subcore_barrier()

        def fire_load(k, blk):
            off = pl.multiple_of((base + blk) * EB, EB)
            pltpu.async_copy(col_hbm.at[pl.ds(off, EB)], idx[k], lsem[k])

        def wait_load(k, blk):
            off = pl.multiple_of((base + blk) * EB, EB)
            pltpu.make_async_copy(col_hbm.at[pl.ds(off, EB)], idx[k], lsem[k]).wait()

        for k in range(GD):
            fire_load(k, k)

        def body(i, carry):
            for k in range(GD):
                wait_load(k, i * GD + k)
                pltpu.async_copy(ones_v, acc_sh.at[idx[k]], ssem[k], add=True)
            for k in range(GD):
                pltpu.make_async_copy(ones_v, acc_sh.at[idx[k]], ssem[k]).wait()

                @pl.when(i < NGRP - 1)
                def _():
                    fire_load(k, (i + 1) * GD + k)
            return carry

        lax.fori_loop(0, NGRP, body, 0)
        off = pl.multiple_of((base + NGRP * GD) * EB, EB)
        pltpu.sync_copy(col_hbm.at[pl.ds(off, EB)], idx[0])
        pltpu.sync_copy(ones_v, acc_sh.at[idx[0]], add=True)

        plsc.subcore_barrier()
        zoff = 0
        while zoff < ZR:
            step = min(EB, ZR - zoff)
            pltpu.sync_copy(acc_sh.at[pl.ds(s * ZR + zoff, step)],
                            ones_v.at[pl.ds(0, step)])
            pltpu.sync_copy(ones_v.at[pl.ds(0, step)],
                            out_hbm.at[c, pl.ds(s * ZR + zoff, step)])
            zoff += step

    return gather2, scatter_add, count


def _sc_gather2(x, idx2):
    return _sc_build()[0](x, idx2)


def _sc_scatter_add(m, col_s, zeros):
    return _sc_build()[1](m, col_s, zeros)


def _sc_count(col_s, ones, zeros):
    return _sc_build()[2](col_s, ones, zeros)


BF16 = jnp.bfloat16
U32 = jnp.uint32
U16 = jnp.uint16


def _unpack2(p):
    u = lax.bitcast_convert_type(p, U32)
    lo = lax.bitcast_convert_type((u & 0xFFFF).astype(U16), BF16)
    hi = lax.bitcast_convert_type((u >> 16).astype(U16), BF16)
    return jnp.concatenate([lo, hi], axis=1)


def _pack2(x16):
    lo = lax.bitcast_convert_type(x16[:, :DP], U16).astype(U32)
    hi = lax.bitcast_convert_type(x16[:, DP:], U16).astype(U32)
    return lax.bitcast_convert_type(lo | (hi << 16), F32)


def _edge_body(xr, xd, ea, wes, wed, wee, be, wn1x, wn1e, bn1, e_ref, m_ref):
    xr16 = _unpack2(xr[...])
    xd16 = _unpack2(xd[...])
    acc = jnp.dot(xr16, wes[...], preferred_element_type=F32)
    acc += jnp.dot(xd16, wed[...], preferred_element_type=F32)
    acc += jnp.dot(ea[...], wee[...], preferred_element_type=F32)
    e16 = jnp.maximum(acc + be[...], 0.0).astype(BF16)
    e_ref[...] = e16
    m = jnp.dot(xr16, wn1x[...], preferred_element_type=F32)
    m += jnp.dot(e16, wn1e[...], preferred_element_type=F32)
    m_ref[...] = jnp.maximum(m + bn1[...], 0.0)


def _tc_edge(xr, xd, ea, wes, wed, wee, be, wn1x, wn1e, bn1):
    ei = ea.shape[1]
    cst = lambda i: (0, 0)
    row = lambda i: (i, 0)
    return pl.pallas_call(
        _edge_body,
        grid=(GE,),
        in_specs=[
            pl.BlockSpec((TE, DP), row),
            pl.BlockSpec((TE, DP), row),
            pl.BlockSpec((TE, ei), row),
            pl.BlockSpec((D, EO), cst),
            pl.BlockSpec((D, EO), cst),
            pl.BlockSpec((ei, EO), cst),
            pl.BlockSpec((1, EO), cst),
            pl.BlockSpec((D, D), cst),
            pl.BlockSpec((EO, D), cst),
            pl.BlockSpec((1, D), cst),
        ],
        out_specs=[pl.BlockSpec((TE, EO), row), pl.BlockSpec((TE, D), row)],
        out_shape=[jax.ShapeDtypeStruct((E_PAD, EO), BF16),
                   jax.ShapeDtypeStruct((E_PAD, D), F32)],
    )(xr, xd, ea, wes, wed, wee, be, wn1x, wn1e, bn1)


def _node_body(x, p0, p1, invc, b3, wn2x, wn2a, bn2, xn_ref, gs_ref, gc_ref):
    i = pl.program_id(0)
    agg = (p0[0] + p1[0]) * invc[...]
    xn = jnp.dot(_unpack2(x[...]), wn2x[...], preferred_element_type=F32)
    xn += jnp.dot(agg, wn2a[...], preferred_element_type=F32)
    xn = jnp.maximum(xn + bn2[...], 0.0)
    xn_ref[...] = _pack2(xn.astype(BF16))
    seg = b3[0]
    ids = lax.broadcasted_iota(I32, (NG, TN), 0)
    oh = (ids == seg).astype(F32)

    @pl.when(i == 0)
    def _init():
        gs_ref[...] = jnp.zeros_like(gs_ref)
        gc_ref[...] = jnp.zeros_like(gc_ref)

    gs_ref[...] += jnp.dot(oh, xn, preferred_element_type=F32)
    gc_ref[...] += jnp.broadcast_to(jnp.sum(oh, axis=1, keepdims=True), (NG, D))


def _tc_node(x, parts, invc, b3, wn2x, wn2a, bn2):
    cst = lambda i: (0, 0)
    return pl.pallas_call(
        _node_body,
        grid=(GN,),
        in_specs=[
            pl.BlockSpec((TN, DP), lambda i: (i, 0)),
            pl.BlockSpec((1, TN, D), lambda i: (0, i, 0)),
            pl.BlockSpec((1, TN, D), lambda i: (1, i, 0)),
            pl.BlockSpec((TN, D), lambda i: (i, 0)),
            pl.BlockSpec((1, 1, TN), lambda i: (i, 0, 0)),
            pl.BlockSpec((D, D), cst),
            pl.BlockSpec((D, D), cst),
            pl.BlockSpec((1, D), cst),
        ],
        out_specs=[pl.BlockSpec((TN, DP), lambda i: (i, 0)),
                   pl.BlockSpec((NG, D), cst),
                   pl.BlockSpec((NG, D), cst)],
        out_shape=[jax.ShapeDtypeStruct((N_PAD, DP), F32),
                   jax.ShapeDtypeStruct((NG, D), F32),
                   jax.ShapeDtypeStruct((NG, D), F32)],
    )(x, parts, parts, invc, b3, wn2x, wn2a, bn2)


def _inv_body(c0, c1, out):
    out[...] = 1.0 / jnp.maximum(c0[0] + c1[0], 1.0)


def _tc_invcnt(cnt_parts):
    return pl.pallas_call(
        _inv_body,
        grid=(GN,),
        in_specs=[pl.BlockSpec((1, TN, D), lambda i: (0, i, 0)),
                  pl.BlockSpec((1, TN, D), lambda i: (1, i, 0))],
        out_specs=pl.BlockSpec((TN, D), lambda i: (i, 0)),
        out_shape=jax.ShapeDtypeStruct((N, D), F32),
    )(cnt_parts, cnt_parts)


def _tc_global(gs, gc, u, wgu, wgm, bg):
    go = wgm.shape[1]

    if u is None:
        def body(gs_r, gc_r, wgm_r, bg_r, out):
            gm = gs_r[...] / jnp.maximum(gc_r[...], 1.0)
            out[...] = jnp.dot(gm, wgm_r[...], preferred_element_type=F32) + bg_r[...]
        args = (gs, gc, wgm, bg)
    else:
        def body(gs_r, gc_r, u_r, wgu_r, wgm_r, bg_r, out):
            gm = gs_r[...] / jnp.maximum(gc_r[...], 1.0)
            acc = jnp.dot(gm, wgm_r[...], preferred_element_type=F32)
            acc += jnp.dot(u_r[...], wgu_r[...], preferred_element_type=F32)
            out[...] = acc + bg_r[...]
        args = (gs, gc, u, wgu, wgm, bg)

    return pl.pallas_call(
        body,
        out_shape=jax.ShapeDtypeStruct((NG, go), F32),
    )(*args)


def _head_body(rg, pg, wr, br, wp, bp, wyr, wyp, by, out):
    a = jnp.dot(rg[...], wr[...], preferred_element_type=F32) + br[...]
    a = jnp.where(a > 0, a, jnp.exp(jnp.minimum(a, 0.0)) - 1.0)
    b = jnp.dot(pg[...], wp[...], preferred_element_type=F32) + bp[...]
    b = jnp.where(b > 0, b, jnp.exp(jnp.minimum(b, 0.0)) - 1.0)
    y = jnp.dot(a, wyr[...], preferred_element_type=F32)
    y += jnp.dot(b, wyp[...], preferred_element_type=F32)
    y = jax.nn.sigmoid(y + by[...]) * 100.0
    out[...] = jnp.broadcast_to(y[:, :1], (NG, D))


def _tc_head(rg, pg, wr, br, wp, bp, wyr, wyp, by):
    return pl.pallas_call(
        _head_body,
        out_shape=jax.ShapeDtypeStruct((NG, D), F32),
    )(rg, pg, wr, br, wp, bp, wyr, wyp, by)


def _meta_layer(x, ea, u, idx2, col_s, invc, b3, zeros, prm):
    (wes, wed, wee, be, wn1x, wn1e, bn1, wn2x, wn2a, bn2, wgu, wgm, bg) = prm
    xr, xd = _sc_gather2(x, idx2)
    e, m = _tc_edge(xr, xd, ea, wes, wed, wee, be, wn1x, wn1e, bn1)
    parts = _sc_scatter_add(m, col_s, zeros)
    xn, gs, gc = _tc_node(x, parts, invc, b3, wn2x, wn2a, bn2)
    un = _tc_global(gs, gc, u, wgu, wgm, bg)
    return xn, e, un


def _split_layer(We, be, Wn1, bn1, Wn2, bn2, Wg, bg, gi, fold_bn=None):
    wes, wed, wee = We[:D], We[D:2 * D], We[2 * D:]
    wn1x, wn1e = Wn1[:D], Wn1[D:]
    wn2x, wn2a = Wn2[:D], Wn2[D:]
    if fold_bn is not None:
        nsc, nsh, esc, esh = fold_bn
        be = be + nsh @ wes + nsh @ wed + esh @ wee
        wes = nsc[:, None] * wes
        wed = nsc[:, None] * wed
        wee = esc[:, None] * wee
        bn1 = bn1 + nsh @ wn1x
        wn1x = nsc[:, None] * wn1x
        bn2 = bn2 + nsh @ wn2x
        wn2x = nsc[:, None] * wn2x
    wgu, wgm = (Wg[:gi], Wg[gi:]) if gi > 0 else (None, Wg)
    b16 = jnp.bfloat16
    return (wes.astype(b16), wed.astype(b16), wee.astype(b16), be[None, :],
            wn1x.astype(b16), wn1e.astype(b16), bn1[None, :],
            wn2x.astype(b16), wn2a, bn2[None, :], wgu, wgm, bg[None, :])


def kernel(rx, re, rg, px, pe, rc, pc, rb, pb, bnn_g, bnn_b, bnn_m, bnn_v, bne_g, bne_b, bne_m, bne_v, We_r1, be_r1, Wn1_r1, bn1_r1, Wn2_r1, bn2_r1, Wg_r1, bg_r1, We_r2, be_r2, Wn1_r2, bn1_r2, Wn2_r2, bn2_r2, Wg_r2, bg_r2, We_r3, be_r3, Wn1_r3, bn1_r3, Wn2_r3, bn2_r3, Wg_r3, bg_r3, We_r4, be_r4, Wn1_r4, bn1_r4, Wn2_r4, bn2_r4, Wg_r4, bg_r4, We_r5, be_r5, Wn1_r5, bn1_r5, Wn2_r5, bn2_r5, Wg_r5, bg_r5, We_p1, be_p1, Wn1_p1, bn1_p1, Wn2_p1, bn2_p1, Wg_p1, bg_p1, We_p2, be_p2, Wn1_p2, bn1_p2, Wn2_p2, bn2_p2, Wg_p2, bg_p2, We_p3, be_p3, Wn1_p3, bn1_p3, Wn2_p3, bn2_p3, Wg_p3, bg_p3, W_rlin, b_rlin, W_plin, b_plin, W_y, b_y):
    pad = E_PAD - E
    zpad = jnp.zeros((pad,), I32)
    npad = jnp.full((pad,), N, I32)
    rrow = jnp.concatenate([rc[0].astype(I32), zpad])
    rcol_g = jnp.concatenate([rc[1].astype(I32), zpad])
    rcol_s = jnp.concatenate([rc[1].astype(I32), npad])
    prow = jnp.concatenate([pc[0].astype(I32), zpad])
    pcol_g = jnp.concatenate([pc[1].astype(I32), zpad])
    pcol_s = jnp.concatenate([pc[1].astype(I32), npad])
    ridx2 = jnp.stack([rrow.reshape(NBLK, EB), rcol_g.reshape(NBLK, EB)], axis=1)
    pidx2 = jnp.stack([prow.reshape(NBLK, EB), pcol_g.reshape(NBLK, EB)], axis=1)
    rb3 = rb.astype(I32).reshape(GN, 1, TN)
    pb3 = pb.astype(I32).reshape(GN, 1, TN)

    nsc = bnn_g / jnp.sqrt(bnn_v + EPS)
    nsh = bnn_b - bnn_m * nsc
    esc = bne_g / jnp.sqrt(bne_v + EPS)
    esh = bne_b - bne_m * esc
    fold = (nsc, nsh, esc, esh)

    re_p = jnp.concatenate([re, jnp.zeros((pad, re.shape[1]), F32)]).astype(jnp.bfloat16)
    pe_p = jnp.concatenate([pe, jnp.zeros((pad, pe.shape[1]), F32)]).astype(jnp.bfloat16)

    zeros = jnp.zeros((EB, D), F32)
    ones = jnp.ones((EB, D), F32)

    def pack_host(x):
        x16 = x.astype(jnp.bfloat16)
        lo = lax.bitcast_convert_type(x16[:, :DP], jnp.uint16).astype(jnp.uint32)
        hi = lax.bitcast_convert_type(x16[:, DP:], jnp.uint16).astype(jnp.uint32)
        return lax.bitcast_convert_type(lo | (hi << 16), F32)

    rx_p = jnp.pad(pack_host(rx), ((0, N_PAD - N), (0, 0)))
    px_p = jnp.pad(pack_host(px), ((0, N_PAD - N), (0, 0)))

    r_layers = [
        _split_layer(We_r1, be_r1, Wn1_r1, bn1_r1, Wn2_r1, bn2_r1, Wg_r1, bg_r1, 32, fold),
        _split_layer(We_r2, be_r2, Wn1_r2, bn1_r2, Wn2_r2, bn2_r2, Wg_r2, bg_r2, 128),
        _split_layer(We_r3, be_r3, Wn1_r3, bn1_r3, Wn2_r3, bn2_r3, Wg_r3, bg_r3, 128),
        _split_layer(We_r4, be_r4, Wn1_r4, bn1_r4, Wn2_r4, bn2_r4, Wg_r4, bg_r4, 128),
        _split_layer(We_r5, be_r5, Wn1_r5, bn1_r5, Wn2_r5, bn2_r5, Wg_r5, bg_r5, 128),
    ]
    p_layers = [
        _split_layer(We_p1, be_p1, Wn1_p1, bn1_p1, Wn2_p1, bn2_p1, Wg_p1, bg_p1, 0, fold),
        _split_layer(We_p2, be_p2, Wn1_p2, bn1_p2, Wn2_p2, bn2_p2, Wg_p2, bg_p2, 128),
        _split_layer(We_p3, be_p3, Wn1_p3, bn1_p3, Wn2_p3, bn2_p3, Wg_p3, bg_p3, 128),
    ]

    r_invc = _tc_invcnt(_sc_count(rcol_s, ones, zeros))
    p_invc = _tc_invcnt(_sc_count(pcol_s, ones, zeros))

    rst = (rx_p, re_p, rg)
    pst = (px_p, pe_p, None)
    for i in range(5):
        rst = _meta_layer(*rst, ridx2, rcol_s, r_invc, rb3, zeros, r_layers[i])
        if i < 3:
            pst = _meta_layer(*pst, pidx2, pcol_s, p_invc, pb3, zeros, p_layers[i])
    rg_out = rst[2]
    pg_out = pst[2]

    y = _tc_head(rg_out, pg_out,
                 W_rlin, b_rlin[None, :], W_plin, b_plin[None, :],
                 W_y[:256], W_y[256:], b_y[None, :])
    return y[:, :1]
